# Initial kernel scaffold; baseline (speedup 1.0000x reference)
import jax
import jax.numpy as jnp
from jax.experimental import pallas as pl


def kernel(x, convW0, convb0, poolW0, poolb0, convW1, convb1, poolW1, poolb1, convW2, convb2, poolW2, poolb2, lin1W, lin1b, lin2W, lin2b, lin3W, lin3b, edge_index):
    raise NotImplementedError("write your pallas kernel here")


# speedup vs baseline: 1.0078x; 1.0078x over previous
import functools
import math

import numpy as np
import jax
import jax.numpy as jnp
from jax import lax
from jax.experimental import pallas as pl
from jax.experimental.pallas import tpu as pltpu
from jax.experimental.pallas import tpu_sc as plsc

_INTERPRET = False

N = 10000
NP = 10240
E = 160000
D = 256
RB = 1024
NEG_INF = np.float32(-np.inf)


def _mm_body(feat_ref, norm_ref, w_ref, out_ref):
    f = feat_ref[...] * norm_ref[...]
    out_ref[...] = jnp.dot(f, w_ref[...], preferred_element_type=jnp.float32)


def _scaled_mm(feat, norm_out, w):
    return pl.pallas_call(
        _mm_body,
        grid=(NP // RB,),
        in_specs=[
            pl.BlockSpec((RB, D), lambda i: (i, 0)),
            pl.BlockSpec((RB, 1), lambda i: (i, 0)),
            pl.BlockSpec((D, D), lambda i: (0, 0)),
        ],
        out_specs=pl.BlockSpec((RB, D), lambda i: (i, 0)),
        out_shape=jax.ShapeDtypeStruct((NP, D), jnp.float32),
        interpret=_INTERPRET,
    )(feat, norm_out, w)


def _ep_body(agg_ref, ni_ref, b_ref, m_ref, wp_ref, no_ref, feat_ref, s_ref):
    out = (agg_ref[...] * ni_ref[...] + b_ref[...]) * m_ref[...]
    f = jnp.maximum(out, 0.0)
    feat_ref[...] = f
    s_ref[...] = jnp.sum(f * wp_ref[...], axis=1, keepdims=True) * no_ref[...]


def _epilogue(agg, norm_in, b, mask, wp_row, norm_out):
    return pl.pallas_call(
        _ep_body,
        grid=(NP // RB,),
        in_specs=[
            pl.BlockSpec((RB, D), lambda i: (i, 0)),
            pl.BlockSpec((RB, 1), lambda i: (i, 0)),
            pl.BlockSpec((1, D), lambda i: (0, 0)),
            pl.BlockSpec((RB, 1), lambda i: (i, 0)),
            pl.BlockSpec((1, D), lambda i: (0, 0)),
            pl.BlockSpec((RB, 1), lambda i: (i, 0)),
        ],
        out_specs=[
            pl.BlockSpec((RB, D), lambda i: (i, 0)),
            pl.BlockSpec((RB, 1), lambda i: (i, 0)),
        ],
        out_shape=[
            jax.ShapeDtypeStruct((NP, D), jnp.float32),
            jax.ShapeDtypeStruct((NP, 1), jnp.float32),
        ],
        interpret=_INTERPRET,
    )(agg, norm_in, b, mask, wp_row, norm_out)


def _top_body(k, aggs_ref, ni_ref, bp_ref, m_ref, nm_ref, mult_ref):
    score = aggs_ref[...] * ni_ref[...] + bp_ref[0, 0]
    sm = jnp.where(m_ref[...] > 0, score, NEG_INF)
    ub = lax.bitcast_convert_type(sm, jnp.uint32)
    top = jnp.uint32(0x80000000)
    u = jnp.where(ub >= top, ~ub, ub | top)

    def bit_body(i, p):
        cand = p | (jnp.uint32(1) << (jnp.uint32(31) - i))
        cnt = jnp.sum((u >= cand).astype(jnp.int32))
        return jnp.where(cnt >= k, cand, p)

    t = lax.fori_loop(0, 32, bit_body, jnp.uint32(0))
    c_gt = jnp.sum((u > t).astype(jnp.int32))
    mrem = k - c_gt
    ties = u == t
    idx = (lax.broadcasted_iota(jnp.int32, (NP // 128, 128), 0) * 128
           + lax.broadcasted_iota(jnp.int32, (NP // 128, 128), 1))

    def tie_body(i, c):
        cand = c + (jnp.int32(1) << (jnp.int32(13) - i))
        cnt = jnp.sum((ties & (idx < cand)).astype(jnp.int32))
        return jnp.where(cnt <= mrem, cand, c)

    cstar = lax.fori_loop(0, 14, tie_body, jnp.int32(0))
    nm = ((u > t) | (ties & (idx < cstar))).astype(jnp.float32)
    nm_ref[...] = nm
    mult_ref[...] = jnp.tanh(sm) * nm


def _sag_topk(aggs80, ni80, bp, mask80, k):
    return pl.pallas_call(
        functools.partial(_top_body, k),
        out_shape=[
            jax.ShapeDtypeStruct((NP // 128, 128), jnp.float32),
            jax.ShapeDtypeStruct((NP // 128, 128), jnp.float32),
        ],
        interpret=_INTERPRET,
    )(aggs80, ni80, bp, mask80)


def _apply_body(k, feat_ref, mult_ref, nm_ref, fo_ref, ro_ref):
    i = pl.program_id(0)
    fn = feat_ref[...] * mult_ref[...]
    fo_ref[...] = fn

    @pl.when(i == 0)
    def _():
        ro_ref[...] = jnp.full((2, D), NEG_INF, jnp.float32)
        ro_ref[0:1, :] = jnp.zeros((1, D), jnp.float32)

    ro_ref[0:1, :] += jnp.sum(fn, axis=0, keepdims=True)
    ro_ref[1:2, :] = jnp.maximum(
        ro_ref[1:2, :],
        jnp.max(jnp.where(nm_ref[...] > 0, fn, NEG_INF), axis=0,
                keepdims=True))

    @pl.when(i == NP // RB - 1)
    def _():
        ro_ref[0:1, :] = ro_ref[0:1, :] / jnp.float32(k)


def _apply_pool(feat, mult, nmask, k):
    return pl.pallas_call(
        functools.partial(_apply_body, k),
        grid=(NP // RB,),
        in_specs=[
            pl.BlockSpec((RB, D), lambda i: (i, 0)),
            pl.BlockSpec((RB, 1), lambda i: (i, 0)),
            pl.BlockSpec((RB, 1), lambda i: (i, 0)),
        ],
        out_specs=[
            pl.BlockSpec((RB, D), lambda i: (i, 0)),
            pl.BlockSpec((2, D), lambda i: (0, 0)),
        ],
        out_shape=[
            jax.ShapeDtypeStruct((NP, D), jnp.float32),
            jax.ShapeDtypeStruct((2, D), jnp.float32),
        ],
        interpret=_INTERPRET,
    )(feat, mult, nmask)


def _mlp_body(ro_ref, w1_ref, b1_ref, w2_ref, b2_ref, w3_ref, b3_ref, o_ref):
    r = ro_ref[0] + ro_ref[1] + ro_ref[2]
    avg = r[0:1, :]
    mx = r[1:2, :]
    h = avg @ w1_ref[0:D, :] + mx @ w1_ref[D:2 * D, :] + b1_ref[...]
    h = jnp.maximum(h, 0.0)
    h = jnp.maximum(h @ w2_ref[...] + b2_ref[...], 0.0)
    logits = h @ w3_ref[...] + b3_ref[...]
    m = jnp.max(logits)
    o_ref[...] = logits - (m + jnp.log(jnp.sum(jnp.exp(logits - m))))


def _mlp(ros, w1, b1, w2, b2, w3, b3):
    return pl.pallas_call(
        _mlp_body,
        out_shape=jax.ShapeDtypeStruct((1, 10), jnp.float32),
        interpret=_INTERPRET,
    )(ros, w1, b1, w2, b2, w3, b3)


def _degrees(mask, src, dst):
    em = mask[src] * mask[dst]
    deg_o = jax.ops.segment_sum(em, src, num_segments=NP)
    deg_i = jax.ops.segment_sum(em, dst, num_segments=NP)
    return deg_o, deg_i


def _aggregate(hs, src, dst):
    return jax.ops.segment_sum(hs[src], dst, num_segments=NP)


def _aggregate_scalar(s, src, dst):
    return jax.ops.segment_sum(s[src, 0], dst, num_segments=NP)[:, None]


def kernel(x, convW0, convb0, poolW0, poolb0, convW1, convb1, poolW1, poolb1,
           convW2, convb2, poolW2, poolb2, lin1W, lin1b, lin2W, lin2b,
           lin3W, lin3b, edge_index):
    src = edge_index[0]
    dst = edge_index[1]
    feat = jnp.pad(x, ((0, NP - N), (0, 0)))
    mask = (jnp.arange(NP) < N).astype(jnp.float32)

    layers = [(convW0, convb0, poolW0, poolb0),
              (convW1, convb1, poolW1, poolb1),
              (convW2, convb2, poolW2, poolb2)]
    ksizes = [5000, 2500, 1250]
    ros = []
    for (wc, bc, wp, bp), k in zip(layers, ksizes):
        deg_o, deg_i = _degrees(mask, src, dst)
        norm_o = jnp.where(deg_o > 0, deg_o ** -0.5, 0.0)[:, None]
        norm_i = jnp.where(deg_i > 0, deg_i ** -0.5, 0.0)[:, None]
        hs = _scaled_mm(feat, norm_o, wc)
        agg = _aggregate(hs, src, dst)
        feat, s_scaled = _epilogue(agg, norm_i, bc[None, :], mask[:, None],
                                   wp.reshape(1, D), norm_o)
        aggs = _aggregate_scalar(s_scaled, src, dst)
        nm80, mult80 = _sag_topk(aggs.reshape(NP // 128, 128),
                                 norm_i.reshape(NP // 128, 128),
                                 bp.reshape(1, 1),
                                 mask.reshape(NP // 128, 128), k)
        nmask = nm80.reshape(NP, 1)
        feat, ro = _apply_pool(feat, mult80.reshape(NP, 1), nmask, k)
        ros.append(ro)
        mask = nmask[:, 0]
    return _mlp(jnp.stack(ros), lin1W, lin1b[None, :], lin2W, lin2b[None, :],
                lin3W, lin3b[None, :])


# speedup vs baseline: 18.0789x; 17.9393x over previous
import functools
import math

import numpy as np
import jax
import jax.numpy as jnp
from jax import lax
from jax.experimental import pallas as pl
from jax.experimental.pallas import tpu as pltpu
from jax.experimental.pallas import tpu_sc as plsc

_INTERPRET = False

N = 10000
NP = 10240
E = 160000
D = 256
RB = 1024
NEG_INF = np.float32(-np.inf)


def _mm_body(feat_ref, norm_ref, w_ref, out_ref):
    f = feat_ref[...] * norm_ref[...]
    out_ref[0] = jnp.dot(f, w_ref[...], preferred_element_type=jnp.float32)


def _scaled_mm(feat, norm_out, w):
    return pl.pallas_call(
        _mm_body,
        grid=(NP // RB, 2),
        in_specs=[
            pl.BlockSpec((RB, D), lambda i, j: (i, 0)),
            pl.BlockSpec((RB, 1), lambda i, j: (i, 0)),
            pl.BlockSpec((D, 128), lambda i, j: (0, j)),
        ],
        out_specs=pl.BlockSpec((1, RB, 128), lambda i, j: (j, i, 0)),
        out_shape=jax.ShapeDtypeStruct((2, NP, 128), jnp.float32),
        interpret=_INTERPRET,
    )(feat, norm_out, w)


def _ep_body(agg_ref, ni_ref, b_ref, m_ref, wp_ref, no_ref, feat_ref, s_ref):
    agg = jnp.concatenate([agg_ref[0], agg_ref[1]], axis=1)
    out = (agg * ni_ref[...] + b_ref[...]) * m_ref[...]
    f = jnp.maximum(out, 0.0)
    feat_ref[...] = f
    s_ref[...] = jnp.sum(f * wp_ref[...], axis=1, keepdims=True) * no_ref[...]


def _epilogue(agg, norm_in, b, mask, wp_row, norm_out):
    return pl.pallas_call(
        _ep_body,
        grid=(NP // RB,),
        in_specs=[
            pl.BlockSpec((2, RB, 128), lambda i: (0, i, 0)),
            pl.BlockSpec((RB, 1), lambda i: (i, 0)),
            pl.BlockSpec((1, D), lambda i: (0, 0)),
            pl.BlockSpec((RB, 1), lambda i: (i, 0)),
            pl.BlockSpec((1, D), lambda i: (0, 0)),
            pl.BlockSpec((RB, 1), lambda i: (i, 0)),
        ],
        out_specs=[
            pl.BlockSpec((RB, D), lambda i: (i, 0)),
            pl.BlockSpec((RB, 1), lambda i: (i, 0)),
        ],
        out_shape=[
            jax.ShapeDtypeStruct((NP, D), jnp.float32),
            jax.ShapeDtypeStruct((NP, 1), jnp.float32),
        ],
        interpret=_INTERPRET,
    )(agg, norm_in, b, mask, wp_row, norm_out)


def _top_body(k, aggs_ref, ni_ref, bp_ref, m_ref, nm_ref, mult_ref):
    score = aggs_ref[...] * ni_ref[...] + bp_ref[0, 0]
    sm = jnp.where(m_ref[...] > 0, score, NEG_INF)
    ub = lax.bitcast_convert_type(sm, jnp.uint32)
    top = jnp.uint32(0x80000000)
    u = jnp.where(ub >= top, ~ub, ub | top)

    def bit_body(i, p):
        cand = p | (jnp.uint32(1) << (jnp.uint32(31) - i))
        cnt = jnp.sum((u >= cand).astype(jnp.int32))
        return jnp.where(cnt >= k, cand, p)

    t = lax.fori_loop(0, 32, bit_body, jnp.uint32(0))
    c_gt = jnp.sum((u > t).astype(jnp.int32))
    mrem = k - c_gt
    ties = u == t
    idx = (lax.broadcasted_iota(jnp.int32, (NP // 128, 128), 0) * 128
           + lax.broadcasted_iota(jnp.int32, (NP // 128, 128), 1))

    def tie_body(i, c):
        cand = c + (jnp.int32(1) << (jnp.int32(13) - i))
        cnt = jnp.sum((ties & (idx < cand)).astype(jnp.int32))
        return jnp.where(cnt <= mrem, cand, c)

    cstar = lax.fori_loop(0, 14, tie_body, jnp.int32(0))
    nm = ((u > t) | (ties & (idx < cstar))).astype(jnp.float32)
    nm_ref[...] = nm
    mult_ref[...] = jnp.tanh(sm) * nm


def _sag_topk(aggs80, ni80, bp, mask80, k):
    return pl.pallas_call(
        functools.partial(_top_body, k),
        out_shape=[
            jax.ShapeDtypeStruct((NP // 128, 128), jnp.float32),
            jax.ShapeDtypeStruct((NP // 128, 128), jnp.float32),
        ],
        interpret=_INTERPRET,
    )(aggs80, ni80, bp, mask80)


def _apply_body(k, feat_ref, mult_ref, nm_ref, fo_ref, ro_ref):
    i = pl.program_id(0)
    fn = feat_ref[...] * mult_ref[...]
    fo_ref[...] = fn

    @pl.when(i == 0)
    def _():
        ro_ref[...] = jnp.full((2, D), NEG_INF, jnp.float32)
        ro_ref[0:1, :] = jnp.zeros((1, D), jnp.float32)

    ro_ref[0:1, :] += jnp.sum(fn, axis=0, keepdims=True)
    ro_ref[1:2, :] = jnp.maximum(
        ro_ref[1:2, :],
        jnp.max(jnp.where(nm_ref[...] > 0, fn, NEG_INF), axis=0,
                keepdims=True))

    @pl.when(i == NP // RB - 1)
    def _():
        ro_ref[0:1, :] = ro_ref[0:1, :] / jnp.float32(k)


def _apply_pool(feat, mult, nmask, k):
    return pl.pallas_call(
        functools.partial(_apply_body, k),
        grid=(NP // RB,),
        in_specs=[
            pl.BlockSpec((RB, D), lambda i: (i, 0)),
            pl.BlockSpec((RB, 1), lambda i: (i, 0)),
            pl.BlockSpec((RB, 1), lambda i: (i, 0)),
        ],
        out_specs=[
            pl.BlockSpec((RB, D), lambda i: (i, 0)),
            pl.BlockSpec((2, D), lambda i: (0, 0)),
        ],
        out_shape=[
            jax.ShapeDtypeStruct((NP, D), jnp.float32),
            jax.ShapeDtypeStruct((2, D), jnp.float32),
        ],
        interpret=_INTERPRET,
    )(feat, mult, nmask)


def _mlp_body(ro_ref, w1_ref, b1_ref, w2_ref, b2_ref, w3_ref, b3_ref, o_ref):
    r = ro_ref[0] + ro_ref[1] + ro_ref[2]
    avg = r[0:1, :]
    mx = r[1:2, :]
    h = avg @ w1_ref[0:D, :] + mx @ w1_ref[D:2 * D, :] + b1_ref[...]
    h = jnp.maximum(h, 0.0)
    h = jnp.maximum(h @ w2_ref[...] + b2_ref[...], 0.0)
    logits = h @ w3_ref[...] + b3_ref[...]
    m = jnp.max(logits)
    o_ref[...] = logits - (m + jnp.log(jnp.sum(jnp.exp(logits - m))))


def _mlp(ros, w1, b1, w2, b2, w3, b3):
    return pl.pallas_call(
        _mlp_body,
        out_shape=jax.ShapeDtypeStruct((1, 10), jnp.float32),
        interpret=_INTERPRET,
    )(ros, w1, b1, w2, b2, w3, b3)


ECH = 79
EPT = ECH * 128
EP = 16 * EPT
NPT = NP // 16
LUT = 1024


def _sc_mesh():
    return plsc.VectorSubcoreMesh(core_axis_name="c", subcore_axis_name="s")


def _zero_vec(ref, n2d):
    z = jnp.zeros((16,), jnp.float32)

    def body(i, _):
        for l in range(8):
            ref[i, pl.ds(l * 16, 16)] = z
        return 0

    lax.fori_loop(0, n2d, body, 0)


def _zero_flat(ref, n):
    z = jnp.zeros((16,), jnp.float32)

    def body(i, _):
        ref[pl.ds(i * 16, 16)] = z
        return 0

    lax.fori_loop(0, n, body, 0)


def _deg_body(src_hbm, dst_hbm, mask_hbm, lut_hbm, no_hbm, ni_hbm,
              mask_v, lut_v, srcb, dstb, emb, zb, dgb, nrb,
              dego_sp, degi_sp, sem):
    cid = lax.axis_index("c")
    sid = lax.axis_index("s")
    pltpu.sync_copy(mask_hbm, mask_v)
    pltpu.sync_copy(lut_hbm, lut_v)
    pltpu.sync_copy(src_hbm.at[sid], srcb)
    pltpu.sync_copy(dst_hbm.at[sid], dstb)
    _zero_flat(zb, NPT // 16)
    pltpu.sync_copy(zb, dego_sp.at[pl.ds(sid * NPT, NPT)])
    pltpu.sync_copy(zb, degi_sp.at[pl.ds(sid * NPT, NPT)])
    plsc.subcore_barrier()

    def chunk(j, _):
        for l in range(8):
            s16 = srcb[j, pl.ds(l * 16, 16)]
            d16 = dstb[j, pl.ds(l * 16, 16)]
            em = (plsc.load_gather(mask_v, [s16])
                  * plsc.load_gather(mask_v, [d16]))
            emb[j, pl.ds(l * 16, 16)] = em
        c1 = pltpu.async_copy(emb.at[j], dego_sp.at[srcb.at[j]], sem,
                              add=True)
        c2 = pltpu.async_copy(emb.at[j], degi_sp.at[dstb.at[j]], sem,
                              add=True)
        c1.wait()
        c2.wait()
        return 0

    lax.fori_loop(0, ECH, chunk, 0)
    plsc.subcore_barrier()

    base = cid * (NP // 2) + sid * 320

    def norms(sp_ref, out_hbm):
        pltpu.sync_copy(sp_ref.at[pl.ds(base, 320)], dgb)

        def body(i, _):
            d16 = dgb[pl.ds(i * 16, 16)]
            di = jnp.minimum(d16, np.float32(LUT - 1)).astype(jnp.int32)
            nrb[pl.ds(i * 16, 16)] = plsc.load_gather(lut_v,
                                                      [di])
            return 0

        lax.fori_loop(0, 20, body, 0)
        pltpu.sync_copy(nrb, out_hbm.at[pl.ds(base, 320)])

    norms(dego_sp, no_hbm)
    norms(degi_sp, ni_hbm)


def _sc_degrees(mask, lut, src3, dst3):
    f = pl.kernel(
        _deg_body,
        out_type=[jax.ShapeDtypeStruct((NP,), jnp.float32),
                  jax.ShapeDtypeStruct((NP,), jnp.float32)],
        mesh=_sc_mesh(),
        compiler_params=pltpu.CompilerParams(needs_layout_passes=False),
        scratch_types=[
            pltpu.VMEM((NP,), jnp.float32),
            pltpu.VMEM((LUT,), jnp.float32),
            pltpu.VMEM((ECH, 128), jnp.int32),
            pltpu.VMEM((ECH, 128), jnp.int32),
            pltpu.VMEM((ECH, 128), jnp.float32),
            pltpu.VMEM((NPT,), jnp.float32),
            pltpu.VMEM((320,), jnp.float32),
            pltpu.VMEM((320,), jnp.float32),
            pltpu.VMEM_SHARED((NP,), jnp.float32),
            pltpu.VMEM_SHARED((NP,), jnp.float32),
            pltpu.SemaphoreType.DMA,
        ],
    )
    return f(src3, dst3, mask, lut)


def _agg_body(h_hbm, src_hbm, dst_hbm, out_hbm,
              srcb, dstb, gb, zb, acc_sp, semg, sems):
    cid = lax.axis_index("c")
    sid = lax.axis_index("s")
    pltpu.sync_copy(src_hbm.at[sid], srcb)
    pltpu.sync_copy(dst_hbm.at[sid], dstb)
    _zero_vec(zb, 64)

    def zloop(t, _):
        pltpu.sync_copy(zb, acc_sp.at[pl.ds(sid * NPT + t * 64, 64)])
        return 0

    lax.fori_loop(0, NPT // 64, zloop, 0)
    plsc.subcore_barrier()

    def chunk(j, _):
        pltpu.async_copy(h_hbm.at[cid].at[srcb.at[j]], gb, semg).wait()
        pltpu.async_copy(gb, acc_sp.at[dstb.at[j]], sems, add=True).wait()
        return 0

    lax.fori_loop(0, ECH, chunk, 0)
    plsc.subcore_barrier()

    def wloop(t, _):
        rows = sid * NPT + t * 128
        pltpu.sync_copy(acc_sp.at[pl.ds(rows, 128)], gb)
        pltpu.sync_copy(gb, out_hbm.at[cid].at[pl.ds(rows, 128)])
        return 0

    lax.fori_loop(0, NPT // 128, wloop, 0)


def _sc_aggregate(hsplit, src3, dst3):
    f = pl.kernel(
        _agg_body,
        out_type=jax.ShapeDtypeStruct((2, NP, 128), jnp.float32),
        mesh=_sc_mesh(),
        compiler_params=pltpu.CompilerParams(needs_layout_passes=False),
        scratch_types=[
            pltpu.VMEM((ECH, 128), jnp.int32),
            pltpu.VMEM((ECH, 128), jnp.int32),
            pltpu.VMEM((128, 128), jnp.float32),
            pltpu.VMEM((64, 128), jnp.float32),
            pltpu.VMEM_SHARED((NP, 128), jnp.float32),
            pltpu.SemaphoreType.DMA,
            pltpu.SemaphoreType.DMA,
        ],
    )
    return f(hsplit, src3, dst3)


def _aggs_body(s_hbm, src_hbm, dst_hbm, out_hbm,
               s_v, srcb, dstb, sb, zb, agg_sp, sem):
    cid = lax.axis_index("c")
    sid = lax.axis_index("s")
    pltpu.sync_copy(s_hbm, s_v)
    pltpu.sync_copy(src_hbm.at[sid], srcb)
    pltpu.sync_copy(dst_hbm.at[sid], dstb)
    _zero_flat(zb, NPT // 16)
    pltpu.sync_copy(zb, agg_sp.at[pl.ds(sid * NPT, NPT)])
    plsc.subcore_barrier()

    def chunk(j, _):
        for l in range(8):
            s16 = srcb[j, pl.ds(l * 16, 16)]
            sb[j, pl.ds(l * 16, 16)] = plsc.load_gather(
                s_v, [s16])
        pltpu.async_copy(sb.at[j], agg_sp.at[dstb.at[j]], sem,
                         add=True).wait()
        return 0

    lax.fori_loop(0, ECH, chunk, 0)
    plsc.subcore_barrier()
    base = cid * (NP // 2) + sid * 320
    pltpu.sync_copy(agg_sp.at[pl.ds(base, 320)], zb.at[pl.ds(0, 320)])
    pltpu.sync_copy(zb.at[pl.ds(0, 320)], out_hbm.at[pl.ds(base, 320)])


def _sc_aggregate_scalar(s_flat, src3, dst3):
    f = pl.kernel(
        _aggs_body,
        out_type=jax.ShapeDtypeStruct((NP,), jnp.float32),
        mesh=_sc_mesh(),
        compiler_params=pltpu.CompilerParams(needs_layout_passes=False),
        scratch_types=[
            pltpu.VMEM((NP,), jnp.float32),
            pltpu.VMEM((ECH, 128), jnp.int32),
            pltpu.VMEM((ECH, 128), jnp.int32),
            pltpu.VMEM((ECH, 128), jnp.float32),
            pltpu.VMEM((NPT,), jnp.float32),
            pltpu.VMEM_SHARED((NP,), jnp.float32),
            pltpu.SemaphoreType.DMA,
        ],
    )
    return f(s_flat, src3, dst3)


def kernel(x, convW0, convb0, poolW0, poolb0, convW1, convb1, poolW1, poolb1,
           convW2, convb2, poolW2, poolb2, lin1W, lin1b, lin2W, lin2b,
           lin3W, lin3b, edge_index):
    pad_idx = N + (jnp.arange(EP - E, dtype=jnp.int32) % (NP - N))
    src3 = jnp.concatenate([edge_index[0], pad_idx]).reshape(16, ECH, 128)
    dst3 = jnp.concatenate([edge_index[1], pad_idx]).reshape(16, ECH, 128)
    lut_d = jnp.arange(LUT, dtype=jnp.float32)
    lut = jnp.where(lut_d > 0, jnp.where(lut_d > 0, lut_d, 1.0) ** -0.5, 0.0)
    feat = jnp.pad(x, ((0, NP - N), (0, 0)))
    mask = (jnp.arange(NP) < N).astype(jnp.float32)

    layers = [(convW0, convb0, poolW0, poolb0),
              (convW1, convb1, poolW1, poolb1),
              (convW2, convb2, poolW2, poolb2)]
    ksizes = [5000, 2500, 1250]
    ros = []
    for (wc, bc, wp, bp), k in zip(layers, ksizes):
        norm_o, norm_i = _sc_degrees(mask, lut, src3, dst3)
        hs = _scaled_mm(feat, norm_o[:, None], wc)
        agg = _sc_aggregate(hs, src3, dst3)
        feat, s_scaled = _epilogue(agg, norm_i[:, None], bc[None, :],
                                   mask[:, None], wp.reshape(1, D),
                                   norm_o[:, None])
        aggs = _sc_aggregate_scalar(s_scaled.reshape(NP), src3, dst3)
        nm80, mult80 = _sag_topk(aggs.reshape(NP // 128, 128),
                                 norm_i.reshape(NP // 128, 128),
                                 bp.reshape(1, 1),
                                 mask.reshape(NP // 128, 128), k)
        nmask = nm80.reshape(NP, 1)
        feat, ro = _apply_pool(feat, mult80.reshape(NP, 1), nmask, k)
        ros.append(ro)
        mask = nmask[:, 0]
    return _mlp(jnp.stack(ros), lin1W, lin1b[None, :], lin2W, lin2b[None, :],
                lin3W, lin3b[None, :])


# speedup vs baseline: 21.4338x; 1.1856x over previous
import functools
import math

import numpy as np
import jax
import jax.numpy as jnp
from jax import lax
from jax.experimental import pallas as pl
from jax.experimental.pallas import tpu as pltpu
from jax.experimental.pallas import tpu_sc as plsc

_INTERPRET = False

N = 10000
NP = 10240
E = 160000
D = 256
RB = 1024
NEG_INF = np.float32(-np.inf)


def _mm_body(feat_ref, mult_ref, norm_ref, w_ref, out_ref):
    f = feat_ref[...] * (mult_ref[...] * norm_ref[...])
    out_ref[0] = jnp.dot(f, w_ref[...], preferred_element_type=jnp.float32)


def _scaled_mm(feat, mult, norm_out, w):
    return pl.pallas_call(
        _mm_body,
        grid=(NP // RB, 2),
        in_specs=[
            pl.BlockSpec((RB, D), lambda i, j: (i, 0)),
            pl.BlockSpec((RB, 1), lambda i, j: (i, 0)),
            pl.BlockSpec((RB, 1), lambda i, j: (i, 0)),
            pl.BlockSpec((D, 128), lambda i, j: (0, j)),
        ],
        out_specs=pl.BlockSpec((1, RB, 128), lambda i, j: (j, i, 0)),
        out_shape=jax.ShapeDtypeStruct((2, NP, 128), jnp.float32),
        interpret=_INTERPRET,
    )(feat, mult, norm_out, w)


def _ep_body(agg_ref, ni_ref, b_ref, m_ref, wp_ref, no_ref, feat_ref, s_ref):
    agg = jnp.concatenate([agg_ref[0], agg_ref[1]], axis=1)
    out = (agg * ni_ref[...] + b_ref[...]) * m_ref[...]
    f = jnp.maximum(out, 0.0)
    feat_ref[...] = f
    s_ref[...] = jnp.sum(f * wp_ref[...], axis=1, keepdims=True) * no_ref[...]


def _epilogue(agg, norm_in, b, mask, wp_row, norm_out):
    return pl.pallas_call(
        _ep_body,
        grid=(NP // RB,),
        in_specs=[
            pl.BlockSpec((2, RB, 128), lambda i: (0, i, 0)),
            pl.BlockSpec((RB, 1), lambda i: (i, 0)),
            pl.BlockSpec((1, D), lambda i: (0, 0)),
            pl.BlockSpec((RB, 1), lambda i: (i, 0)),
            pl.BlockSpec((1, D), lambda i: (0, 0)),
            pl.BlockSpec((RB, 1), lambda i: (i, 0)),
        ],
        out_specs=[
            pl.BlockSpec((RB, D), lambda i: (i, 0)),
            pl.BlockSpec((RB, 1), lambda i: (i, 0)),
        ],
        out_shape=[
            jax.ShapeDtypeStruct((NP, D), jnp.float32),
            jax.ShapeDtypeStruct((NP, 1), jnp.float32),
        ],
        interpret=_INTERPRET,
    )(agg, norm_in, b, mask, wp_row, norm_out)


def _top_body(k, aggs_ref, ni_ref, bp_ref, m_ref, nm_ref, mult_ref):
    score = aggs_ref[...] * ni_ref[...] + bp_ref[0, 0]
    sm = jnp.where(m_ref[...] > 0, score, NEG_INF)
    ub = lax.bitcast_convert_type(sm, jnp.uint32)
    top = jnp.uint32(0x80000000)
    u = jnp.where(ub >= top, ~ub, ub | top)

    def bit_body(i, p):
        cand = p | (jnp.uint32(1) << (jnp.uint32(31) - i))
        cnt = jnp.sum((u >= cand).astype(jnp.int32))
        return jnp.where(cnt >= k, cand, p)

    t = lax.fori_loop(0, 32, bit_body, jnp.uint32(0))
    c_gt = jnp.sum((u > t).astype(jnp.int32))
    mrem = k - c_gt
    ties = u == t
    idx = (lax.broadcasted_iota(jnp.int32, (NP // 128, 128), 0) * 128
           + lax.broadcasted_iota(jnp.int32, (NP // 128, 128), 1))

    def tie_body(i, c):
        cand = c + (jnp.int32(1) << (jnp.int32(13) - i))
        cnt = jnp.sum((ties & (idx < cand)).astype(jnp.int32))
        return jnp.where(cnt <= mrem, cand, c)

    cstar = lax.fori_loop(0, 14, tie_body, jnp.int32(0))
    nm = ((u > t) | (ties & (idx < cstar))).astype(jnp.float32)
    nm_ref[...] = nm
    mult_ref[...] = jnp.tanh(sm) * nm


def _sag_topk(aggs80, ni80, bp, mask80, k):
    return pl.pallas_call(
        functools.partial(_top_body, k),
        out_shape=[
            jax.ShapeDtypeStruct((NP // 128, 128), jnp.float32),
            jax.ShapeDtypeStruct((NP // 128, 128), jnp.float32),
        ],
        interpret=_INTERPRET,
    )(aggs80, ni80, bp, mask80)


def _apply_body(k, feat_ref, mult_ref, nm_ref, ro_ref):
    i = pl.program_id(0)
    fn = feat_ref[...] * mult_ref[...]

    @pl.when(i == 0)
    def _():
        ro_ref[...] = jnp.full((2, D), NEG_INF, jnp.float32)
        ro_ref[0:1, :] = jnp.zeros((1, D), jnp.float32)

    ro_ref[0:1, :] += jnp.sum(fn, axis=0, keepdims=True)
    ro_ref[1:2, :] = jnp.maximum(
        ro_ref[1:2, :],
        jnp.max(jnp.where(nm_ref[...] > 0, fn, NEG_INF), axis=0,
                keepdims=True))

    @pl.when(i == NP // RB - 1)
    def _():
        ro_ref[0:1, :] = ro_ref[0:1, :] / jnp.float32(k)


def _apply_pool(feat, mult, nmask, k):
    return pl.pallas_call(
        functools.partial(_apply_body, k),
        grid=(NP // RB,),
        in_specs=[
            pl.BlockSpec((RB, D), lambda i: (i, 0)),
            pl.BlockSpec((RB, 1), lambda i: (i, 0)),
            pl.BlockSpec((RB, 1), lambda i: (i, 0)),
        ],
        out_specs=pl.BlockSpec((2, D), lambda i: (0, 0)),
        out_shape=jax.ShapeDtypeStruct((2, D), jnp.float32),
        interpret=_INTERPRET,
    )(feat, mult, nmask)


def _mlp_body(ro_ref, w1_ref, b1_ref, w2_ref, b2_ref, w3_ref, b3_ref, o_ref):
    r = ro_ref[0] + ro_ref[1] + ro_ref[2]
    avg = r[0:1, :]
    mx = r[1:2, :]
    h = avg @ w1_ref[0:D, :] + mx @ w1_ref[D:2 * D, :] + b1_ref[...]
    h = jnp.maximum(h, 0.0)
    h = jnp.maximum(h @ w2_ref[...] + b2_ref[...], 0.0)
    logits = h @ w3_ref[...] + b3_ref[...]
    m = jnp.max(logits)
    o_ref[...] = logits - (m + jnp.log(jnp.sum(jnp.exp(logits - m))))


def _mlp(ros, w1, b1, w2, b2, w3, b3):
    return pl.pallas_call(
        _mlp_body,
        out_shape=jax.ShapeDtypeStruct((1, 10), jnp.float32),
        interpret=_INTERPRET,
    )(ros, w1, b1, w2, b2, w3, b3)


ECH = 79
ECHA = 158
EPT = ECH * 128
EP = 16 * EPT
NPT = NP // 16
LUT = 1024


def _sc_mesh():
    return plsc.VectorSubcoreMesh(core_axis_name="c", subcore_axis_name="s")


def _zero_vec(ref, n2d):
    z = jnp.zeros((16,), jnp.float32)

    def body(i, _):
        for l in range(8):
            ref[i, pl.ds(l * 16, 16)] = z
        return 0

    lax.fori_loop(0, n2d, body, 0)


def _zero_flat(ref, n):
    z = jnp.zeros((16,), jnp.float32)

    def body(i, _):
        ref[pl.ds(i * 16, 16)] = z
        return 0

    lax.fori_loop(0, n, body, 0)


def _deg_body(src_hbm, dst_hbm, mask_hbm, lut_hbm, no_hbm, ni_hbm,
              mask_v, lut_v, srcb, dstb, emb, zb, dgb, nrb,
              dego_sp, degi_sp, sem):
    cid = lax.axis_index("c")
    sid = lax.axis_index("s")
    pltpu.sync_copy(mask_hbm, mask_v)
    pltpu.sync_copy(lut_hbm, lut_v)
    pltpu.sync_copy(src_hbm.at[sid], srcb)
    pltpu.sync_copy(dst_hbm.at[sid], dstb)
    _zero_flat(zb, NPT // 16)
    pltpu.sync_copy(zb, dego_sp.at[pl.ds(sid * NPT, NPT)])
    pltpu.sync_copy(zb, degi_sp.at[pl.ds(sid * NPT, NPT)])
    plsc.subcore_barrier()

    def chunk(j, _):
        for l in range(8):
            s16 = srcb[j, pl.ds(l * 16, 16)]
            d16 = dstb[j, pl.ds(l * 16, 16)]
            em = (plsc.load_gather(mask_v, [s16])
                  * plsc.load_gather(mask_v, [d16]))
            emb[j, pl.ds(l * 16, 16)] = em
        c1 = pltpu.async_copy(emb.at[j], dego_sp.at[srcb.at[j]], sem,
                              add=True)
        c2 = pltpu.async_copy(emb.at[j], degi_sp.at[dstb.at[j]], sem,
                              add=True)
        c1.wait()
        c2.wait()
        return 0

    lax.fori_loop(0, ECH, chunk, 0)
    plsc.subcore_barrier()

    base = cid * (NP // 2) + sid * 320

    def norms(sp_ref, out_hbm):
        pltpu.sync_copy(sp_ref.at[pl.ds(base, 320)], dgb)

        def body(i, _):
            d16 = dgb[pl.ds(i * 16, 16)]
            di = jnp.minimum(d16, np.float32(LUT - 1)).astype(jnp.int32)
            nrb[pl.ds(i * 16, 16)] = plsc.load_gather(lut_v,
                                                      [di])
            return 0

        lax.fori_loop(0, 20, body, 0)
        pltpu.sync_copy(nrb, out_hbm.at[pl.ds(base, 320)])

    norms(dego_sp, no_hbm)
    norms(degi_sp, ni_hbm)


def _sc_degrees(mask, lut, src3, dst3):
    f = pl.kernel(
        _deg_body,
        out_type=[jax.ShapeDtypeStruct((NP,), jnp.float32),
                  jax.ShapeDtypeStruct((NP,), jnp.float32)],
        mesh=_sc_mesh(),
        compiler_params=pltpu.CompilerParams(needs_layout_passes=False),
        scratch_types=[
            pltpu.VMEM((NP,), jnp.float32),
            pltpu.VMEM((LUT,), jnp.float32),
            pltpu.VMEM((ECH, 128), jnp.int32),
            pltpu.VMEM((ECH, 128), jnp.int32),
            pltpu.VMEM((ECH, 128), jnp.float32),
            pltpu.VMEM((NPT,), jnp.float32),
            pltpu.VMEM((320,), jnp.float32),
            pltpu.VMEM((320,), jnp.float32),
            pltpu.VMEM_SHARED((NP,), jnp.float32),
            pltpu.VMEM_SHARED((NP,), jnp.float32),
            pltpu.SemaphoreType.DMA,
        ],
    )
    return f(src3, dst3, mask, lut)


def _agg_body(h_hbm, src_hbm, dst_hbm, out_hbm,
              ring, dstb, gbuf, acc_sp, semi, semg, sems):
    cid = lax.axis_index("c")
    sid = lax.axis_index("s")
    pltpu.sync_copy(dst_hbm.at[sid], dstb)
    _zero_vec(gbuf.at[0], 128)

    def zloop(t, _):
        pltpu.sync_copy(gbuf.at[0], acc_sp.at[pl.ds(sid * NPT + t * 128, 128)])
        return 0

    lax.fori_loop(0, NPT // 128, zloop, 0)
    plsc.subcore_barrier()

    def si(j):
        return pltpu.async_copy(src_hbm.at[sid].at[j], ring.at[j % 4], semi)

    def wi(j):
        pltpu.make_async_copy(src_hbm.at[sid].at[j], ring.at[j % 4],
                              semi).wait()

    def sg(j):
        return pltpu.async_copy(h_hbm.at[cid].at[ring.at[j % 4]],
                                gbuf.at[j % 2], semg)

    def wg(j):
        pltpu.make_async_copy(h_hbm.at[cid].at[ring.at[j % 4]],
                              gbuf.at[j % 2], semg).wait()

    def ss(j):
        return pltpu.async_copy(gbuf.at[j % 2], acc_sp.at[dstb.at[j]], sems,
                                add=True)

    def ws(j):
        pltpu.make_async_copy(gbuf.at[j % 2], acc_sp.at[dstb.at[j]],
                              sems).wait()

    si(0).wait()
    sg(0)

    def chunk(j, _):
        @pl.when(j + 1 < ECH)
        def _():
            si(j + 1)

        wg(j)

        @pl.when(j + 1 < ECH)
        def _():
            wi(j + 1)

            @pl.when(j >= 1)
            def _():
                ws(j - 1)

            sg(j + 1)

        ss(j)
        return 0

    lax.fori_loop(0, ECH, chunk, 0)
    ws(ECH - 2)
    ws(ECH - 1)
    plsc.subcore_barrier()

    def wloop(t, _):
        rows = sid * NPT + t * 128
        pltpu.sync_copy(acc_sp.at[pl.ds(rows, 128)], gbuf.at[0])
        pltpu.sync_copy(gbuf.at[0], out_hbm.at[cid].at[pl.ds(rows, 128)])
        return 0

    lax.fori_loop(0, NPT // 128, wloop, 0)


def _sc_aggregate(hsplit, src3, dst3):
    f = pl.kernel(
        _agg_body,
        out_type=jax.ShapeDtypeStruct((2, NP, 128), jnp.float32),
        mesh=_sc_mesh(),
        compiler_params=pltpu.CompilerParams(needs_layout_passes=False),
        scratch_types=[
            pltpu.VMEM((4, 128), jnp.int32),
            pltpu.VMEM((ECH, 128), jnp.int32),
            pltpu.VMEM((2, 128, 128), jnp.float32),
            pltpu.VMEM_SHARED((NP, 128), jnp.float32),
            pltpu.SemaphoreType.DMA,
            pltpu.SemaphoreType.DMA,
            pltpu.SemaphoreType.DMA,
        ],
    )
    return f(hsplit, src3, dst3)


def _aggs_body(s_hbm, src_hbm, dst_hbm, out_hbm,
               s_v, srcb, dstb, sb, zb, agg_sp, sem):
    cid = lax.axis_index("c")
    sid = lax.axis_index("s")
    pltpu.sync_copy(s_hbm, s_v)
    pltpu.sync_copy(src_hbm.at[sid], srcb)
    pltpu.sync_copy(dst_hbm.at[sid], dstb)
    _zero_flat(zb, NPT // 16)
    pltpu.sync_copy(zb, agg_sp.at[pl.ds(sid * NPT, NPT)])
    plsc.subcore_barrier()

    def chunk(j, _):
        for l in range(8):
            s16 = srcb[j, pl.ds(l * 16, 16)]
            sb[j, pl.ds(l * 16, 16)] = plsc.load_gather(
                s_v, [s16])
        pltpu.async_copy(sb.at[j], agg_sp.at[dstb.at[j]], sem,
                         add=True).wait()
        return 0

    lax.fori_loop(0, ECH, chunk, 0)
    plsc.subcore_barrier()
    base = cid * (NP // 2) + sid * 320
    pltpu.sync_copy(agg_sp.at[pl.ds(base, 320)], zb.at[pl.ds(0, 320)])
    pltpu.sync_copy(zb.at[pl.ds(0, 320)], out_hbm.at[pl.ds(base, 320)])


def _sc_aggregate_scalar(s_flat, src3, dst3):
    f = pl.kernel(
        _aggs_body,
        out_type=jax.ShapeDtypeStruct((NP,), jnp.float32),
        mesh=_sc_mesh(),
        compiler_params=pltpu.CompilerParams(needs_layout_passes=False),
        scratch_types=[
            pltpu.VMEM((NP,), jnp.float32),
            pltpu.VMEM((ECH, 128), jnp.int32),
            pltpu.VMEM((ECH, 128), jnp.int32),
            pltpu.VMEM((ECH, 128), jnp.float32),
            pltpu.VMEM((NPT,), jnp.float32),
            pltpu.VMEM_SHARED((NP,), jnp.float32),
            pltpu.SemaphoreType.DMA,
        ],
    )
    return f(s_flat, src3, dst3)


def kernel(x, convW0, convb0, poolW0, poolb0, convW1, convb1, poolW1, poolb1,
           convW2, convb2, poolW2, poolb2, lin1W, lin1b, lin2W, lin2b,
           lin3W, lin3b, edge_index):
    pad_idx = N + (jnp.arange(EP - E, dtype=jnp.int32) % (NP - N))
    src_pad = jnp.concatenate([edge_index[0], pad_idx])
    dst_pad = jnp.concatenate([edge_index[1], pad_idx])
    src3 = src_pad.reshape(16, ECH, 128)
    dst3 = dst_pad.reshape(16, ECH, 128)
    lut_d = jnp.arange(LUT, dtype=jnp.float32)
    lut = jnp.where(lut_d > 0, jnp.where(lut_d > 0, lut_d, 1.0) ** -0.5, 0.0)
    feat = jnp.pad(x, ((0, NP - N), (0, 0)))
    mask = (jnp.arange(NP) < N).astype(jnp.float32)
    mult = jnp.ones((NP, 1), jnp.float32)

    layers = [(convW0, convb0, poolW0, poolb0),
              (convW1, convb1, poolW1, poolb1),
              (convW2, convb2, poolW2, poolb2)]
    ksizes = [5000, 2500, 1250]
    ros = []
    for (wc, bc, wp, bp), k in zip(layers, ksizes):
        norm_o, norm_i = _sc_degrees(mask, lut, src3, dst3)
        hs = _scaled_mm(feat, mult, norm_o[:, None], wc)
        agg = _sc_aggregate(hs, src3, dst3)
        feat, s_scaled = _epilogue(agg, norm_i[:, None], bc[None, :],
                                   mask[:, None], wp.reshape(1, D),
                                   norm_o[:, None])
        aggs = _sc_aggregate_scalar(s_scaled.reshape(NP), src3, dst3)
        nm80, mult80 = _sag_topk(aggs.reshape(NP // 128, 128),
                                 norm_i.reshape(NP // 128, 128),
                                 bp.reshape(1, 1),
                                 mask.reshape(NP // 128, 128), k)
        nmask = nm80.reshape(NP, 1)
        mult = mult80.reshape(NP, 1)
        ro = _apply_pool(feat, mult, nmask, k)
        ros.append(ro)
        mask = nmask[:, 0]
    return _mlp(jnp.stack(ros), lin1W, lin1b[None, :], lin2W, lin2b[None, :],
                lin3W, lin3b[None, :])


# speedup vs baseline: 28.4909x; 1.3293x over previous
import functools
import math

import numpy as np
import jax
import jax.numpy as jnp
from jax import lax
from jax.experimental import pallas as pl
from jax.experimental.pallas import tpu as pltpu
from jax.experimental.pallas import tpu_sc as plsc

_INTERPRET = False

N = 10000
NP = 10240
E = 160000
D = 256
RB = 1024
NEG_INF = np.float32(-np.inf)


def _mm_body(feat_ref, mult_ref, norm_ref, w_ref, out_ref):
    f = feat_ref[...] * (mult_ref[...] * norm_ref[...])
    out_ref[0] = jnp.dot(f, w_ref[...], preferred_element_type=jnp.float32)


def _scaled_mm(feat, mult, norm_out, w):
    return pl.pallas_call(
        _mm_body,
        grid=(NP // RB, 2),
        in_specs=[
            pl.BlockSpec((RB, D), lambda i, j: (i, 0)),
            pl.BlockSpec((RB, 1), lambda i, j: (i, 0)),
            pl.BlockSpec((RB, 1), lambda i, j: (i, 0)),
            pl.BlockSpec((D, 128), lambda i, j: (0, j)),
        ],
        out_specs=pl.BlockSpec((1, RB, 128), lambda i, j: (j, i, 0)),
        out_shape=jax.ShapeDtypeStruct((2, NP, 128), jnp.float32),
        interpret=_INTERPRET,
    )(feat, mult, norm_out, w)


def _ep_body(agg_ref, ni_ref, b_ref, m_ref, wp_ref, no_ref, feat_ref, s_ref):
    agg = jnp.concatenate([agg_ref[0], agg_ref[1]], axis=1)
    out = (agg * ni_ref[...] + b_ref[...]) * m_ref[...]
    f = jnp.maximum(out, 0.0)
    feat_ref[...] = f
    s_ref[...] = jnp.sum(f * wp_ref[...], axis=1, keepdims=True) * no_ref[...]


def _epilogue(agg, norm_in, b, mask, wp_row, norm_out):
    return pl.pallas_call(
        _ep_body,
        grid=(NP // RB,),
        in_specs=[
            pl.BlockSpec((2, RB, 128), lambda i: (0, i, 0)),
            pl.BlockSpec((RB, 1), lambda i: (i, 0)),
            pl.BlockSpec((1, D), lambda i: (0, 0)),
            pl.BlockSpec((RB, 1), lambda i: (i, 0)),
            pl.BlockSpec((1, D), lambda i: (0, 0)),
            pl.BlockSpec((RB, 1), lambda i: (i, 0)),
        ],
        out_specs=[
            pl.BlockSpec((RB, D), lambda i: (i, 0)),
            pl.BlockSpec((RB, 1), lambda i: (i, 0)),
        ],
        out_shape=[
            jax.ShapeDtypeStruct((NP, D), jnp.float32),
            jax.ShapeDtypeStruct((NP, 1), jnp.float32),
        ],
        interpret=_INTERPRET,
    )(agg, norm_in, b, mask, wp_row, norm_out)


def _top_body(k, aggs_ref, ni_ref, bp_ref, m_ref, nm_ref, mult_ref):
    score = aggs_ref[...] * ni_ref[...] + bp_ref[0, 0]
    sm = jnp.where(m_ref[...] > 0, score, NEG_INF)
    ub = lax.bitcast_convert_type(sm, jnp.uint32)
    top = jnp.uint32(0x80000000)
    u = jnp.where(ub >= top, ~ub, ub | top)

    def bit_body(i, p):
        cand = p | (jnp.uint32(1) << (jnp.uint32(31) - i))
        cnt = jnp.sum((u >= cand).astype(jnp.int32))
        return jnp.where(cnt >= k, cand, p)

    t = lax.fori_loop(0, 32, bit_body, jnp.uint32(0))
    c_gt = jnp.sum((u > t).astype(jnp.int32))
    mrem = k - c_gt
    ties = u == t
    idx = (lax.broadcasted_iota(jnp.int32, (NP // 128, 128), 0) * 128
           + lax.broadcasted_iota(jnp.int32, (NP // 128, 128), 1))

    def tie_body(i, c):
        cand = c + (jnp.int32(1) << (jnp.int32(13) - i))
        cnt = jnp.sum((ties & (idx < cand)).astype(jnp.int32))
        return jnp.where(cnt <= mrem, cand, c)

    cstar = lax.fori_loop(0, 14, tie_body, jnp.int32(0))
    nm = ((u > t) | (ties & (idx < cstar))).astype(jnp.float32)
    nm_ref[...] = nm
    mult_ref[...] = jnp.tanh(sm) * nm


def _sag_topk(aggs80, ni80, bp, mask80, k):
    return pl.pallas_call(
        functools.partial(_top_body, k),
        out_shape=[
            jax.ShapeDtypeStruct((NP // 128, 128), jnp.float32),
            jax.ShapeDtypeStruct((NP // 128, 128), jnp.float32),
        ],
        interpret=_INTERPRET,
    )(aggs80, ni80, bp, mask80)


def _apply_body(k, feat_ref, mult_ref, nm_ref, ro_ref):
    i = pl.program_id(0)
    fn = feat_ref[...] * mult_ref[...]

    @pl.when(i == 0)
    def _():
        ro_ref[...] = jnp.full((2, D), NEG_INF, jnp.float32)
        ro_ref[0:1, :] = jnp.zeros((1, D), jnp.float32)

    ro_ref[0:1, :] += jnp.sum(fn, axis=0, keepdims=True)
    ro_ref[1:2, :] = jnp.maximum(
        ro_ref[1:2, :],
        jnp.max(jnp.where(nm_ref[...] > 0, fn, NEG_INF), axis=0,
                keepdims=True))

    @pl.when(i == NP // RB - 1)
    def _():
        ro_ref[0:1, :] = ro_ref[0:1, :] / jnp.float32(k)


def _apply_pool(feat, mult, nmask, k):
    return pl.pallas_call(
        functools.partial(_apply_body, k),
        grid=(NP // RB,),
        in_specs=[
            pl.BlockSpec((RB, D), lambda i: (i, 0)),
            pl.BlockSpec((RB, 1), lambda i: (i, 0)),
            pl.BlockSpec((RB, 1), lambda i: (i, 0)),
        ],
        out_specs=pl.BlockSpec((2, D), lambda i: (0, 0)),
        out_shape=jax.ShapeDtypeStruct((2, D), jnp.float32),
        interpret=_INTERPRET,
    )(feat, mult, nmask)


def _mlp_body(ro_ref, w1_ref, b1_ref, w2_ref, b2_ref, w3_ref, b3_ref, o_ref):
    r = ro_ref[0] + ro_ref[1] + ro_ref[2]
    avg = r[0:1, :]
    mx = r[1:2, :]
    h = avg @ w1_ref[0:D, :] + mx @ w1_ref[D:2 * D, :] + b1_ref[...]
    h = jnp.maximum(h, 0.0)
    h = jnp.maximum(h @ w2_ref[...] + b2_ref[...], 0.0)
    logits = h @ w3_ref[...] + b3_ref[...]
    m = jnp.max(logits)
    o_ref[...] = logits - (m + jnp.log(jnp.sum(jnp.exp(logits - m))))


def _mlp(ros, w1, b1, w2, b2, w3, b3):
    return pl.pallas_call(
        _mlp_body,
        out_shape=jax.ShapeDtypeStruct((1, 10), jnp.float32),
        interpret=_INTERPRET,
    )(ros, w1, b1, w2, b2, w3, b3)


ECH = 79
ECHA = 158
EPT = ECH * 128
EP = 16 * EPT
NPT = NP // 16
LUT = 1024


def _sc_mesh():
    return plsc.VectorSubcoreMesh(core_axis_name="c", subcore_axis_name="s")


def _zero_vec(ref, n2d):
    z = jnp.zeros((16,), jnp.float32)

    def body(i, _):
        for l in range(8):
            ref[i, pl.ds(l * 16, 16)] = z
        return 0

    lax.fori_loop(0, n2d, body, 0)


def _zero_flat(ref, n):
    z = jnp.zeros((16,), jnp.float32)

    def body(i, _):
        ref[pl.ds(i * 16, 16)] = z
        return 0

    lax.fori_loop(0, n, body, 0)


def _compact_body(src_hbm, dst_hbm, mask_hbm, lut_hbm,
                  no_hbm, ni_hbm, srcc_hbm, dstc_hbm, nch_hbm,
                  mask_v, lut_v, srcb, dstb, srcf, dstf, onesb, ncb, zb,
                  dgb, nrb, dego_sp, degi_sp, sem):
    cid = lax.axis_index("c")
    sid = lax.axis_index("s")
    pltpu.sync_copy(mask_hbm, mask_v)
    pltpu.sync_copy(lut_hbm, lut_v)
    pltpu.sync_copy(src_hbm.at[sid], srcb)
    pltpu.sync_copy(dst_hbm.at[sid], dstb)
    _zero_flat(zb, NPT // 16)
    _zero_flat(onesb, 8)

    def ofill(i, _):
        onesb[pl.ds(i * 16, 16)] = jnp.ones((16,), jnp.float32)
        return 0

    lax.fori_loop(0, 8, ofill, 0)
    pltpu.sync_copy(zb, dego_sp.at[pl.ds(sid * NPT, NPT)])
    pltpu.sync_copy(zb, degi_sp.at[pl.ds(sid * NPT, NPT)])
    plsc.subcore_barrier()

    def chunk(j, off):
        for l in range(8):
            s16 = srcb[j, pl.ds(l * 16, 16)]
            d16 = dstb[j, pl.ds(l * 16, 16)]
            keep = (plsc.load_gather(mask_v, [s16])
                    * plsc.load_gather(mask_v, [d16])) > 0.0
            plsc.store_compressed(srcf.at[pl.ds(off, 16)], s16, mask=keep)
            plsc.store_compressed(dstf.at[pl.ds(off, 16)], d16, mask=keep)
            off = off + plsc.all_reduce_population_count(keep)[0]
        return off

    off = lax.fori_loop(0, ECH, chunk, 0)
    nch = jnp.maximum((off + 127) // 128, 1)
    end_e = nch * 128
    lanes = lax.iota(jnp.int32, 16)

    def padloop(w, _):
        base = off + w * 16
        dv = N + lax.rem(base, 224) + lanes
        srcf[pl.ds(base, 16)] = dv
        dstf[pl.ds(base, 16)] = dv
        return 0

    lax.fori_loop(0, (end_e - off + 15) // 16, padloop, 0)

    def wrow(j, _):
        pltpu.async_copy(srcf.at[pl.ds(j * 128, 128)],
                         srcc_hbm.at[cid].at[sid].at[j], sem)
        pltpu.async_copy(dstf.at[pl.ds(j * 128, 128)],
                         dstc_hbm.at[cid].at[sid].at[j], sem)
        return 0

    lax.fori_loop(0, nch, wrow, 0)

    def drow(j, _):
        pltpu.make_async_copy(srcf.at[pl.ds(j * 128, 128)],
                              srcc_hbm.at[cid].at[sid].at[j], sem).wait()
        pltpu.make_async_copy(dstf.at[pl.ds(j * 128, 128)],
                              dstc_hbm.at[cid].at[sid].at[j], sem).wait()
        return 0

    lax.fori_loop(0, nch, drow, 0)
    ncb[pl.ds(0, 16)] = jnp.broadcast_to(nch, (16,)).astype(jnp.int32)
    pltpu.sync_copy(ncb, nch_hbm.at[cid].at[sid])

    pltpu.sync_copy(srcc_hbm.at[cid].at[sid], srcb)
    pltpu.sync_copy(dstc_hbm.at[cid].at[sid], dstb)

    def dchunk(j, _):
        c1 = pltpu.async_copy(onesb, dego_sp.at[srcb.at[j]], sem, add=True)
        c2 = pltpu.async_copy(onesb, degi_sp.at[dstb.at[j]], sem, add=True)
        c1.wait()
        c2.wait()
        return 0

    lax.fori_loop(0, nch, dchunk, 0)
    plsc.subcore_barrier()

    base = cid * (NP // 2) + sid * 320

    def norms(sp_ref, out_hbm):
        pltpu.sync_copy(sp_ref.at[pl.ds(base, 320)], dgb)

        def body(i, _):
            d16 = dgb[pl.ds(i * 16, 16)]
            di = jnp.minimum(d16, np.float32(LUT - 1)).astype(jnp.int32)
            nrb[pl.ds(i * 16, 16)] = plsc.load_gather(lut_v, [di])
            return 0

        lax.fori_loop(0, 20, body, 0)
        pltpu.sync_copy(nrb, out_hbm.at[pl.ds(base, 320)])

    norms(dego_sp, no_hbm)
    norms(degi_sp, ni_hbm)


def _sc_compact(mask, lut, src3, dst3):
    f = pl.kernel(
        _compact_body,
        out_type=[jax.ShapeDtypeStruct((NP,), jnp.float32),
                  jax.ShapeDtypeStruct((NP,), jnp.float32),
                  jax.ShapeDtypeStruct((2, 16, ECH, 128), jnp.int32),
                  jax.ShapeDtypeStruct((2, 16, ECH, 128), jnp.int32),
                  jax.ShapeDtypeStruct((2, 16, 16), jnp.int32)],
        mesh=_sc_mesh(),
        compiler_params=pltpu.CompilerParams(needs_layout_passes=False),
        scratch_types=[
            pltpu.VMEM((NP,), jnp.float32),
            pltpu.VMEM((LUT,), jnp.float32),
            pltpu.VMEM((ECH, 128), jnp.int32),
            pltpu.VMEM((ECH, 128), jnp.int32),
            pltpu.VMEM((EPT + 32,), jnp.int32),
            pltpu.VMEM((EPT + 32,), jnp.int32),
            pltpu.VMEM((128,), jnp.float32),
            pltpu.VMEM((16,), jnp.int32),
            pltpu.VMEM((NPT,), jnp.float32),
            pltpu.VMEM((320,), jnp.float32),
            pltpu.VMEM((320,), jnp.float32),
            pltpu.VMEM_SHARED((NP,), jnp.float32),
            pltpu.VMEM_SHARED((NP,), jnp.float32),
            pltpu.SemaphoreType.DMA,
        ],
    )
    return f(src3, dst3, mask, lut)


def _agg_body(h_hbm, src_hbm, dst_hbm, nch_hbm, out_hbm,
              ring, dstb, ncb, gbuf, acc_sp, semi, semg, sems):
    cid = lax.axis_index("c")
    sid = lax.axis_index("s")
    pltpu.sync_copy(dst_hbm.at[cid].at[sid], dstb)
    pltpu.sync_copy(nch_hbm.at[cid].at[sid], ncb)
    nch = ncb[pl.ds(0, 16)][0]
    _zero_vec(gbuf.at[0], 128)

    def zloop(t, _):
        pltpu.sync_copy(gbuf.at[0], acc_sp.at[pl.ds(sid * NPT + t * 128, 128)])
        return 0

    lax.fori_loop(0, NPT // 128, zloop, 0)
    plsc.subcore_barrier()

    def si(j):
        return pltpu.async_copy(src_hbm.at[cid].at[sid].at[j],
                                ring.at[j % 4], semi)

    def wi(j):
        pltpu.make_async_copy(src_hbm.at[cid].at[sid].at[j],
                              ring.at[j % 4], semi).wait()

    def sg(j):
        return pltpu.async_copy(h_hbm.at[cid].at[ring.at[j % 4]],
                                gbuf.at[j % 2], semg)

    def wg(j):
        pltpu.make_async_copy(h_hbm.at[cid].at[ring.at[j % 4]],
                              gbuf.at[j % 2], semg).wait()

    def ss(j):
        return pltpu.async_copy(gbuf.at[j % 2], acc_sp.at[dstb.at[j]], sems,
                                add=True)

    def ws(j):
        pltpu.make_async_copy(gbuf.at[j % 2], acc_sp.at[dstb.at[j]],
                              sems).wait()

    si(0).wait()
    sg(0)

    def chunk(j, _):
        @pl.when(j + 1 < nch)
        def _():
            si(j + 1)

        wg(j)

        @pl.when(j + 1 < nch)
        def _():
            wi(j + 1)

            @pl.when(j >= 1)
            def _():
                ws(j - 1)

            sg(j + 1)

        ss(j)
        return 0

    lax.fori_loop(0, nch, chunk, 0)

    @pl.when(nch >= 2)
    def _():
        ws(nch - 2)

    ws(nch - 1)
    plsc.subcore_barrier()

    def wloop(t, _):
        rows = sid * NPT + t * 128
        pltpu.sync_copy(acc_sp.at[pl.ds(rows, 128)], gbuf.at[0])
        pltpu.sync_copy(gbuf.at[0], out_hbm.at[cid].at[pl.ds(rows, 128)])
        return 0

    lax.fori_loop(0, NPT // 128, wloop, 0)


def _sc_aggregate(hsplit, srcc, dstc, nchs):
    f = pl.kernel(
        _agg_body,
        out_type=jax.ShapeDtypeStruct((2, NP, 128), jnp.float32),
        mesh=_sc_mesh(),
        compiler_params=pltpu.CompilerParams(needs_layout_passes=False),
        scratch_types=[
            pltpu.VMEM((4, 128), jnp.int32),
            pltpu.VMEM((ECH, 128), jnp.int32),
            pltpu.VMEM((16,), jnp.int32),
            pltpu.VMEM((2, 128, 128), jnp.float32),
            pltpu.VMEM_SHARED((NP, 128), jnp.float32),
            pltpu.SemaphoreType.DMA,
            pltpu.SemaphoreType.DMA,
            pltpu.SemaphoreType.DMA,
        ],
    )
    return f(hsplit, srcc, dstc, nchs)


def _aggs_body(s_hbm, src_hbm, dst_hbm, nch_hbm, out_hbm,
               s_v, srcb, dstb, ncb, sb, zb, agg_sp, sem):
    cid = lax.axis_index("c")
    sid = lax.axis_index("s")
    pltpu.sync_copy(s_hbm, s_v)
    pltpu.sync_copy(src_hbm.at[cid].at[sid], srcb)
    pltpu.sync_copy(dst_hbm.at[cid].at[sid], dstb)
    pltpu.sync_copy(nch_hbm.at[cid].at[sid], ncb)
    nch = ncb[pl.ds(0, 16)][0]
    _zero_flat(zb, NPT // 16)
    pltpu.sync_copy(zb, agg_sp.at[pl.ds(sid * NPT, NPT)])
    plsc.subcore_barrier()

    def chunk(j, _):
        for l in range(8):
            s16 = srcb[j, pl.ds(l * 16, 16)]
            sb[j, pl.ds(l * 16, 16)] = plsc.load_gather(
                s_v, [s16])
        pltpu.async_copy(sb.at[j], agg_sp.at[dstb.at[j]], sem,
                         add=True).wait()
        return 0

    lax.fori_loop(0, nch, chunk, 0)
    plsc.subcore_barrier()
    base = cid * (NP // 2) + sid * 320
    pltpu.sync_copy(agg_sp.at[pl.ds(base, 320)], zb.at[pl.ds(0, 320)])
    pltpu.sync_copy(zb.at[pl.ds(0, 320)], out_hbm.at[pl.ds(base, 320)])


def _sc_aggregate_scalar(s_flat, srcc, dstc, nchs):
    f = pl.kernel(
        _aggs_body,
        out_type=jax.ShapeDtypeStruct((NP,), jnp.float32),
        mesh=_sc_mesh(),
        compiler_params=pltpu.CompilerParams(needs_layout_passes=False),
        scratch_types=[
            pltpu.VMEM((NP,), jnp.float32),
            pltpu.VMEM((ECH, 128), jnp.int32),
            pltpu.VMEM((ECH, 128), jnp.int32),
            pltpu.VMEM((16,), jnp.int32),
            pltpu.VMEM((ECH, 128), jnp.float32),
            pltpu.VMEM((NPT,), jnp.float32),
            pltpu.VMEM_SHARED((NP,), jnp.float32),
            pltpu.SemaphoreType.DMA,
        ],
    )
    return f(s_flat, srcc, dstc, nchs)


def kernel(x, convW0, convb0, poolW0, poolb0, convW1, convb1, poolW1, poolb1,
           convW2, convb2, poolW2, poolb2, lin1W, lin1b, lin2W, lin2b,
           lin3W, lin3b, edge_index):
    pad_idx = N + (jnp.arange(EP - E, dtype=jnp.int32) % (NP - N))
    src_pad = jnp.concatenate([edge_index[0], pad_idx])
    dst_pad = jnp.concatenate([edge_index[1], pad_idx])
    src3 = src_pad.reshape(16, ECH, 128)
    dst3 = dst_pad.reshape(16, ECH, 128)
    lut_d = jnp.arange(LUT, dtype=jnp.float32)
    lut = jnp.where(lut_d > 0, jnp.where(lut_d > 0, lut_d, 1.0) ** -0.5, 0.0)
    feat = jnp.pad(x, ((0, NP - N), (0, 0)))
    mask = (jnp.arange(NP) < N).astype(jnp.float32)
    mult = jnp.ones((NP, 1), jnp.float32)

    layers = [(convW0, convb0, poolW0, poolb0),
              (convW1, convb1, poolW1, poolb1),
              (convW2, convb2, poolW2, poolb2)]
    ksizes = [5000, 2500, 1250]
    ros = []
    for (wc, bc, wp, bp), k in zip(layers, ksizes):
        norm_o, norm_i, srcc, dstc, nchs = _sc_compact(mask, lut, src3,
                                                        dst3)
        hs = _scaled_mm(feat, mult, norm_o[:, None], wc)
        agg = _sc_aggregate(hs, srcc, dstc, nchs)
        feat, s_scaled = _epilogue(agg, norm_i[:, None], bc[None, :],
                                   mask[:, None], wp.reshape(1, D),
                                   norm_o[:, None])
        aggs = _sc_aggregate_scalar(s_scaled.reshape(NP), srcc, dstc,
                                    nchs)
        nm80, mult80 = _sag_topk(aggs.reshape(NP // 128, 128),
                                 norm_i.reshape(NP // 128, 128),
                                 bp.reshape(1, 1),
                                 mask.reshape(NP // 128, 128), k)
        nmask = nm80.reshape(NP, 1)
        mult = mult80.reshape(NP, 1)
        ro = _apply_pool(feat, mult, nmask, k)
        ros.append(ro)
        mask = nmask[:, 0]
    return _mlp(jnp.stack(ros), lin1W, lin1b[None, :], lin2W, lin2b[None, :],
                lin3W, lin3b[None, :])


# speedup vs baseline: 28.5391x; 1.0017x over previous
import functools
import math

import numpy as np
import jax
import jax.numpy as jnp
from jax import lax
from jax.experimental import pallas as pl
from jax.experimental.pallas import tpu as pltpu
from jax.experimental.pallas import tpu_sc as plsc

_INTERPRET = False

N = 10000
NP = 10240
E = 160000
D = 256
RB = 1024
NEG_INF = np.float32(-np.inf)


def _mm_body(feat_ref, mult_ref, norm_ref, w_ref, out_ref):
    f = feat_ref[...] * (mult_ref[...] * norm_ref[...])
    out_ref[0] = jnp.dot(f, w_ref[...], preferred_element_type=jnp.float32)


def _scaled_mm(feat, mult, norm_out, w):
    return pl.pallas_call(
        _mm_body,
        grid=(NP // RB, 2),
        in_specs=[
            pl.BlockSpec((RB, D), lambda i, j: (i, 0)),
            pl.BlockSpec((RB, 1), lambda i, j: (i, 0)),
            pl.BlockSpec((RB, 1), lambda i, j: (i, 0)),
            pl.BlockSpec((D, 128), lambda i, j: (0, j)),
        ],
        out_specs=pl.BlockSpec((1, RB, 128), lambda i, j: (j, i, 0)),
        out_shape=jax.ShapeDtypeStruct((2, NP, 128), jnp.float32),
        interpret=_INTERPRET,
    )(feat, mult, norm_out, w)


def _ep_body(agg_ref, ni_ref, b_ref, m_ref, wp_ref, no_ref, feat_ref, s_ref):
    agg = jnp.concatenate([agg_ref[0], agg_ref[1]], axis=1)
    out = jnp.where(m_ref[...] > 0, agg * ni_ref[...] + b_ref[...], 0.0)
    f = jnp.maximum(out, 0.0)
    feat_ref[...] = f
    s_ref[...] = jnp.sum(f * wp_ref[...], axis=1, keepdims=True) * no_ref[...]


def _epilogue(agg, norm_in, b, mask, wp_row, norm_out):
    return pl.pallas_call(
        _ep_body,
        grid=(NP // RB,),
        in_specs=[
            pl.BlockSpec((2, RB, 128), lambda i: (0, i, 0)),
            pl.BlockSpec((RB, 1), lambda i: (i, 0)),
            pl.BlockSpec((1, D), lambda i: (0, 0)),
            pl.BlockSpec((RB, 1), lambda i: (i, 0)),
            pl.BlockSpec((1, D), lambda i: (0, 0)),
            pl.BlockSpec((RB, 1), lambda i: (i, 0)),
        ],
        out_specs=[
            pl.BlockSpec((RB, D), lambda i: (i, 0)),
            pl.BlockSpec((RB, 1), lambda i: (i, 0)),
        ],
        out_shape=[
            jax.ShapeDtypeStruct((NP, D), jnp.float32),
            jax.ShapeDtypeStruct((NP, 1), jnp.float32),
        ],
        interpret=_INTERPRET,
    )(agg, norm_in, b, mask, wp_row, norm_out)


def _top_body(k, aggs_ref, ni_ref, bp_ref, m_ref, nm_ref, mult_ref):
    score = aggs_ref[...] * ni_ref[...] + bp_ref[0, 0]
    sm = jnp.where(m_ref[...] > 0, score, NEG_INF)
    ub = lax.bitcast_convert_type(sm, jnp.uint32)
    top = jnp.uint32(0x80000000)
    u = jnp.where(ub >= top, ~ub, ub | top)

    def bit_body(i, p):
        cand = p | (jnp.uint32(1) << (jnp.uint32(31) - i))
        cnt = jnp.sum((u >= cand).astype(jnp.int32))
        return jnp.where(cnt >= k, cand, p)

    t = lax.fori_loop(0, 32, bit_body, jnp.uint32(0))
    c_gt = jnp.sum((u > t).astype(jnp.int32))
    mrem = k - c_gt
    ties = u == t
    idx = (lax.broadcasted_iota(jnp.int32, (NP // 128, 128), 0) * 128
           + lax.broadcasted_iota(jnp.int32, (NP // 128, 128), 1))

    def tie_body(i, c):
        cand = c + (jnp.int32(1) << (jnp.int32(13) - i))
        cnt = jnp.sum((ties & (idx < cand)).astype(jnp.int32))
        return jnp.where(cnt <= mrem, cand, c)

    cstar = lax.fori_loop(0, 14, tie_body, jnp.int32(0))
    nm = ((u > t) | (ties & (idx < cstar))).astype(jnp.float32)
    nm_ref[...] = nm
    mult_ref[...] = jnp.tanh(sm) * nm


def _sag_topk(aggs80, ni80, bp, mask80, k):
    return pl.pallas_call(
        functools.partial(_top_body, k),
        out_shape=[
            jax.ShapeDtypeStruct((NP // 128, 128), jnp.float32),
            jax.ShapeDtypeStruct((NP // 128, 128), jnp.float32),
        ],
        interpret=_INTERPRET,
    )(aggs80, ni80, bp, mask80)


def _apply_body(k, feat_ref, mult_ref, nm_ref, ro_ref):
    i = pl.program_id(0)
    fn = feat_ref[...] * mult_ref[...]

    @pl.when(i == 0)
    def _():
        ro_ref[...] = jnp.full((2, D), NEG_INF, jnp.float32)
        ro_ref[0:1, :] = jnp.zeros((1, D), jnp.float32)

    ro_ref[0:1, :] += jnp.sum(fn, axis=0, keepdims=True)
    ro_ref[1:2, :] = jnp.maximum(
        ro_ref[1:2, :],
        jnp.max(jnp.where(nm_ref[...] > 0, fn, NEG_INF), axis=0,
                keepdims=True))

    @pl.when(i == NP // RB - 1)
    def _():
        ro_ref[0:1, :] = ro_ref[0:1, :] / jnp.float32(k)


def _apply_pool(feat, mult, nmask, k):
    return pl.pallas_call(
        functools.partial(_apply_body, k),
        grid=(NP // RB,),
        in_specs=[
            pl.BlockSpec((RB, D), lambda i: (i, 0)),
            pl.BlockSpec((RB, 1), lambda i: (i, 0)),
            pl.BlockSpec((RB, 1), lambda i: (i, 0)),
        ],
        out_specs=pl.BlockSpec((2, D), lambda i: (0, 0)),
        out_shape=jax.ShapeDtypeStruct((2, D), jnp.float32),
        interpret=_INTERPRET,
    )(feat, mult, nmask)


def _mlp_body(ro_ref, w1_ref, b1_ref, w2_ref, b2_ref, w3_ref, b3_ref, o_ref):
    r = ro_ref[0] + ro_ref[1] + ro_ref[2]
    avg = r[0:1, :]
    mx = r[1:2, :]
    h = avg @ w1_ref[0:D, :] + mx @ w1_ref[D:2 * D, :] + b1_ref[...]
    h = jnp.maximum(h, 0.0)
    h = jnp.maximum(h @ w2_ref[...] + b2_ref[...], 0.0)
    logits = h @ w3_ref[...] + b3_ref[...]
    m = jnp.max(logits)
    o_ref[...] = logits - (m + jnp.log(jnp.sum(jnp.exp(logits - m))))


def _mlp(ros, w1, b1, w2, b2, w3, b3):
    return pl.pallas_call(
        _mlp_body,
        out_shape=jax.ShapeDtypeStruct((1, 10), jnp.float32),
        interpret=_INTERPRET,
    )(ros, w1, b1, w2, b2, w3, b3)


ECH = 79
ECHA = 158
EPT = ECH * 128
EP = 16 * EPT
NPT = NP // 16
LUT = 1024


def _sc_mesh():
    return plsc.VectorSubcoreMesh(core_axis_name="c", subcore_axis_name="s")


def _zero_vec(ref, n2d):
    z = jnp.zeros((16,), jnp.float32)

    def body(i, _):
        for l in range(8):
            ref[i, pl.ds(l * 16, 16)] = z
        return 0

    lax.fori_loop(0, n2d, body, 0)


def _zero_flat(ref, n):
    z = jnp.zeros((16,), jnp.float32)

    def body(i, _):
        ref[pl.ds(i * 16, 16)] = z
        return 0

    lax.fori_loop(0, n, body, 0)


def _compact_body(src_hbm, dst_hbm, mask_hbm, lut_hbm,
                  no_hbm, ni_hbm, srcc_hbm, dstc_hbm, nch_hbm,
                  mask_v, lut_v, srcb, dstb, srcf, dstf, onesb, ncb, zb,
                  dgb, nrb, dego_sp, degi_sp, sem):
    cid = lax.axis_index("c")
    sid = lax.axis_index("s")
    pltpu.sync_copy(mask_hbm, mask_v)
    pltpu.sync_copy(lut_hbm, lut_v)
    pltpu.sync_copy(src_hbm.at[sid], srcb)
    pltpu.sync_copy(dst_hbm.at[sid], dstb)
    _zero_flat(zb, NPT // 16)
    _zero_flat(onesb, 8)

    def ofill(i, _):
        onesb[pl.ds(i * 16, 16)] = jnp.ones((16,), jnp.float32)
        return 0

    lax.fori_loop(0, 8, ofill, 0)
    pltpu.sync_copy(zb, dego_sp.at[pl.ds(sid * NPT, NPT)])
    pltpu.sync_copy(zb, degi_sp.at[pl.ds(sid * NPT, NPT)])
    plsc.subcore_barrier()

    def chunk(j, off):
        for l in range(8):
            s16 = srcb[j, pl.ds(l * 16, 16)]
            d16 = dstb[j, pl.ds(l * 16, 16)]
            keep = (plsc.load_gather(mask_v, [s16])
                    * plsc.load_gather(mask_v, [d16])) > 0.0
            plsc.store_compressed(srcf.at[pl.ds(off, 16)], s16, mask=keep)
            plsc.store_compressed(dstf.at[pl.ds(off, 16)], d16, mask=keep)
            off = off + plsc.all_reduce_population_count(keep)[0]
        return off

    off = lax.fori_loop(0, ECH, chunk, 0)
    nch = jnp.maximum((off + 127) // 128, 1)
    end_e = nch * 128
    dv = N + lax.iota(jnp.int32, 16)

    def padloop(w, _):
        base = off + w * 16
        srcf[pl.ds(base, 16)] = dv
        dstf[pl.ds(base, 16)] = dv
        return 0

    lax.fori_loop(0, (end_e - off + 15) // 16, padloop, 0)

    def wrow(j, _):
        pltpu.async_copy(srcf.at[pl.ds(j * 128, 128)],
                         srcc_hbm.at[cid].at[sid].at[j], sem)
        pltpu.async_copy(dstf.at[pl.ds(j * 128, 128)],
                         dstc_hbm.at[cid].at[sid].at[j], sem)
        return 0

    lax.fori_loop(0, nch, wrow, 0)

    def drow(j, _):
        pltpu.make_async_copy(srcf.at[pl.ds(j * 128, 128)],
                              srcc_hbm.at[cid].at[sid].at[j], sem).wait()
        pltpu.make_async_copy(dstf.at[pl.ds(j * 128, 128)],
                              dstc_hbm.at[cid].at[sid].at[j], sem).wait()
        return 0

    lax.fori_loop(0, nch, drow, 0)
    ncb[pl.ds(0, 16)] = jnp.broadcast_to(nch, (16,)).astype(jnp.int32)
    pltpu.sync_copy(ncb, nch_hbm.at[cid].at[sid])

    pltpu.sync_copy(srcc_hbm.at[cid].at[sid], srcb)
    pltpu.sync_copy(dstc_hbm.at[cid].at[sid], dstb)

    def dchunk(j, _):
        c1 = pltpu.async_copy(onesb, dego_sp.at[srcb.at[j]], sem, add=True)
        c2 = pltpu.async_copy(onesb, degi_sp.at[dstb.at[j]], sem, add=True)
        c1.wait()
        c2.wait()
        return 0

    lax.fori_loop(0, nch, dchunk, 0)
    plsc.subcore_barrier()

    base = cid * (NP // 2) + sid * 320

    def norms(sp_ref, out_hbm):
        pltpu.sync_copy(sp_ref.at[pl.ds(base, 320)], dgb)

        def body(i, _):
            d16 = dgb[pl.ds(i * 16, 16)]
            di = jnp.minimum(d16, np.float32(LUT - 1)).astype(jnp.int32)
            nrb[pl.ds(i * 16, 16)] = plsc.load_gather(lut_v, [di])
            return 0

        lax.fori_loop(0, 20, body, 0)
        pltpu.sync_copy(nrb, out_hbm.at[pl.ds(base, 320)])

    norms(dego_sp, no_hbm)
    norms(degi_sp, ni_hbm)


def _sc_compact(mask, lut, src3, dst3):
    f = pl.kernel(
        _compact_body,
        out_type=[jax.ShapeDtypeStruct((NP,), jnp.float32),
                  jax.ShapeDtypeStruct((NP,), jnp.float32),
                  jax.ShapeDtypeStruct((2, 16, ECH, 128), jnp.int32),
                  jax.ShapeDtypeStruct((2, 16, ECH, 128), jnp.int32),
                  jax.ShapeDtypeStruct((2, 16, 16), jnp.int32)],
        mesh=_sc_mesh(),
        compiler_params=pltpu.CompilerParams(needs_layout_passes=False),
        scratch_types=[
            pltpu.VMEM((NP,), jnp.float32),
            pltpu.VMEM((LUT,), jnp.float32),
            pltpu.VMEM((ECH, 128), jnp.int32),
            pltpu.VMEM((ECH, 128), jnp.int32),
            pltpu.VMEM((EPT + 32,), jnp.int32),
            pltpu.VMEM((EPT + 32,), jnp.int32),
            pltpu.VMEM((128,), jnp.float32),
            pltpu.VMEM((16,), jnp.int32),
            pltpu.VMEM((NPT,), jnp.float32),
            pltpu.VMEM((320,), jnp.float32),
            pltpu.VMEM((320,), jnp.float32),
            pltpu.VMEM_SHARED((NP,), jnp.float32),
            pltpu.VMEM_SHARED((NP,), jnp.float32),
            pltpu.SemaphoreType.DMA,
        ],
    )
    return f(src3, dst3, mask, lut)


NPA = N + 16


def _agg_body(h_hbm, src_hbm, dst_hbm, nch_hbm, out_hbm,
              sring, dring, ncb, gbuf, acc_sp, semi, semg, sems):
    cid = lax.axis_index("c")
    sid = lax.axis_index("s")
    pltpu.sync_copy(nch_hbm.at[cid].at[sid], ncb)
    nch = ncb[pl.ds(0, 16)][0]
    _zero_vec(gbuf.at[0], 128)

    zb0 = sid * (NPA // 16)

    def zloop(t, _):
        pltpu.sync_copy(gbuf.at[0], acc_sp.at[pl.ds(zb0 + t * 128, 128)])
        return 0

    lax.fori_loop(0, 4, zloop, 0)
    pltpu.sync_copy(gbuf.at[0].at[pl.ds(0, NPA // 16 - 512)],
                    acc_sp.at[pl.ds(zb0 + 512, NPA // 16 - 512)])
    plsc.subcore_barrier()

    def si(j):
        pltpu.async_copy(src_hbm.at[cid].at[sid].at[j], sring.at[j % 3],
                         semi)
        pltpu.async_copy(dst_hbm.at[cid].at[sid].at[j], dring.at[j % 3],
                         semi)

    def wi():
        pltpu.make_async_copy(src_hbm.at[cid].at[sid].at[0], sring.at[0],
                              semi).wait()
        pltpu.make_async_copy(dst_hbm.at[cid].at[sid].at[0], dring.at[0],
                              semi).wait()

    def sg(j):
        pltpu.async_copy(h_hbm.at[cid].at[sring.at[j % 3]], gbuf.at[j % 3],
                         semg)

    def wg():
        pltpu.make_async_copy(h_hbm.at[cid].at[sring.at[0]], gbuf.at[0],
                              semg).wait()

    def ss(j):
        pltpu.async_copy(gbuf.at[j % 3], acc_sp.at[dring.at[j % 3]], sems,
                         add=True)

    def ws():
        pltpu.make_async_copy(gbuf.at[0], acc_sp.at[dring.at[0]],
                              sems).wait()

    si(0)
    wi()
    sg(0)

    @pl.when(nch >= 2)
    def _():
        si(1)

    def chunk(j, _):
        wg()

        @pl.when(j + 1 < nch)
        def _():
            wi()

            @pl.when(j >= 2)
            def _():
                ws()

            sg(j + 1)

        ss(j)

        @pl.when(j + 2 < nch)
        def _():
            si(j + 2)

        return 0

    lax.fori_loop(0, nch, chunk, 0)

    @pl.when(nch >= 2)
    def _():
        ws()

    ws()
    plsc.subcore_barrier()

    _zero_vec(gbuf.at[0], 128)
    wb0 = sid * NPT

    @pl.when(sid < 15)
    def _():
        def wloop(t, _):
            rows = wb0 + t * 128
            pltpu.sync_copy(acc_sp.at[pl.ds(rows, 128)], gbuf.at[1])
            pltpu.sync_copy(gbuf.at[1], out_hbm.at[cid].at[pl.ds(rows, 128)])
            return 0

        lax.fori_loop(0, NPT // 128, wloop, 0)

    @pl.when(sid == 15)
    def _():
        def wloop(t, _):
            rows = 15 * NPT + t * 128
            pltpu.sync_copy(acc_sp.at[pl.ds(rows, 128)], gbuf.at[1])
            pltpu.sync_copy(gbuf.at[1], out_hbm.at[cid].at[pl.ds(rows, 128)])
            return 0

        lax.fori_loop(0, 3, wloop, 0)
        pltpu.sync_copy(acc_sp.at[pl.ds(9984, 32)],
                        gbuf.at[1].at[pl.ds(0, 32)])
        pltpu.sync_copy(gbuf.at[1].at[pl.ds(0, 32)],
                        out_hbm.at[cid].at[pl.ds(9984, 32)])
        pltpu.sync_copy(gbuf.at[0].at[pl.ds(0, 96)],
                        out_hbm.at[cid].at[pl.ds(NPA, 96)])
        pltpu.sync_copy(gbuf.at[0], out_hbm.at[cid].at[pl.ds(10112, 128)])


def _sc_aggregate(hsplit, srcc, dstc, nchs):
    f = pl.kernel(
        _agg_body,
        out_type=jax.ShapeDtypeStruct((2, NP, 128), jnp.float32),
        mesh=_sc_mesh(),
        compiler_params=pltpu.CompilerParams(needs_layout_passes=False),
        scratch_types=[
            pltpu.VMEM((3, 128), jnp.int32),
            pltpu.VMEM((3, 128), jnp.int32),
            pltpu.VMEM((16,), jnp.int32),
            pltpu.VMEM((3, 128, 128), jnp.float32),
            pltpu.VMEM_SHARED((NPA, 128), jnp.float32),
            pltpu.SemaphoreType.DMA,
            pltpu.SemaphoreType.DMA,
            pltpu.SemaphoreType.DMA,
        ],
    )
    return f(hsplit, srcc, dstc, nchs)


def _aggs_body(s_hbm, src_hbm, dst_hbm, nch_hbm, out_hbm,
               s_v, srcb, dstb, ncb, sb, zb, agg_sp, sem):
    cid = lax.axis_index("c")
    sid = lax.axis_index("s")
    pltpu.sync_copy(s_hbm, s_v)
    pltpu.sync_copy(src_hbm.at[cid].at[sid], srcb)
    pltpu.sync_copy(dst_hbm.at[cid].at[sid], dstb)
    pltpu.sync_copy(nch_hbm.at[cid].at[sid], ncb)
    nch = ncb[pl.ds(0, 16)][0]
    _zero_flat(zb, NPT // 16)
    pltpu.sync_copy(zb, agg_sp.at[pl.ds(sid * NPT, NPT)])
    plsc.subcore_barrier()

    def chunk(j, _):
        for l in range(8):
            s16 = srcb[j, pl.ds(l * 16, 16)]
            sb[j, pl.ds(l * 16, 16)] = plsc.load_gather(
                s_v, [s16])
        pltpu.async_copy(sb.at[j], agg_sp.at[dstb.at[j]], sem,
                         add=True).wait()
        return 0

    lax.fori_loop(0, nch, chunk, 0)
    plsc.subcore_barrier()
    base = cid * (NP // 2) + sid * 320
    pltpu.sync_copy(agg_sp.at[pl.ds(base, 320)], zb.at[pl.ds(0, 320)])
    pltpu.sync_copy(zb.at[pl.ds(0, 320)], out_hbm.at[pl.ds(base, 320)])


def _sc_aggregate_scalar(s_flat, srcc, dstc, nchs):
    f = pl.kernel(
        _aggs_body,
        out_type=jax.ShapeDtypeStruct((NP,), jnp.float32),
        mesh=_sc_mesh(),
        compiler_params=pltpu.CompilerParams(needs_layout_passes=False),
        scratch_types=[
            pltpu.VMEM((NP,), jnp.float32),
            pltpu.VMEM((ECH, 128), jnp.int32),
            pltpu.VMEM((ECH, 128), jnp.int32),
            pltpu.VMEM((16,), jnp.int32),
            pltpu.VMEM((ECH, 128), jnp.float32),
            pltpu.VMEM((NPT,), jnp.float32),
            pltpu.VMEM_SHARED((NP,), jnp.float32),
            pltpu.SemaphoreType.DMA,
        ],
    )
    return f(s_flat, srcc, dstc, nchs)


def kernel(x, convW0, convb0, poolW0, poolb0, convW1, convb1, poolW1, poolb1,
           convW2, convb2, poolW2, poolb2, lin1W, lin1b, lin2W, lin2b,
           lin3W, lin3b, edge_index):
    pad_idx = N + (jnp.arange(EP - E, dtype=jnp.int32) % (NP - N))
    src_pad = jnp.concatenate([edge_index[0], pad_idx])
    dst_pad = jnp.concatenate([edge_index[1], pad_idx])
    src3 = src_pad.reshape(16, ECH, 128)
    dst3 = dst_pad.reshape(16, ECH, 128)
    lut_d = jnp.arange(LUT, dtype=jnp.float32)
    lut = jnp.where(lut_d > 0, jnp.where(lut_d > 0, lut_d, 1.0) ** -0.5, 0.0)
    feat = jnp.pad(x, ((0, NP - N), (0, 0)))
    mask = (jnp.arange(NP) < N).astype(jnp.float32)
    mult = jnp.ones((NP, 1), jnp.float32)

    layers = [(convW0, convb0, poolW0, poolb0),
              (convW1, convb1, poolW1, poolb1),
              (convW2, convb2, poolW2, poolb2)]
    ksizes = [5000, 2500, 1250]
    ros = []
    for (wc, bc, wp, bp), k in zip(layers, ksizes):
        norm_o, norm_i, srcc, dstc, nchs = _sc_compact(mask, lut, src3,
                                                        dst3)
        hs = _scaled_mm(feat, mult, norm_o[:, None], wc)
        agg = _sc_aggregate(hs, srcc, dstc, nchs)
        feat, s_scaled = _epilogue(agg, norm_i[:, None], bc[None, :],
                                   mask[:, None], wp.reshape(1, D),
                                   norm_o[:, None])
        aggs = _sc_aggregate_scalar(s_scaled.reshape(NP), srcc, dstc,
                                    nchs)
        nm80, mult80 = _sag_topk(aggs.reshape(NP // 128, 128),
                                 norm_i.reshape(NP // 128, 128),
                                 bp.reshape(1, 1),
                                 mask.reshape(NP // 128, 128), k)
        nmask = nm80.reshape(NP, 1)
        mult = mult80.reshape(NP, 1)
        ro = _apply_pool(feat, mult, nmask, k)
        ros.append(ro)
        mask = nmask[:, 0]
    return _mlp(jnp.stack(ros), lin1W, lin1b[None, :], lin2W, lin2b[None, :],
                lin3W, lin3b[None, :])


# speedup vs baseline: 28.5924x; 1.0019x over previous
import functools
import math

import numpy as np
import jax
import jax.numpy as jnp
from jax import lax
from jax.experimental import pallas as pl
from jax.experimental.pallas import tpu as pltpu
from jax.experimental.pallas import tpu_sc as plsc

_INTERPRET = False

N = 10000
NP = 10240
E = 160000
D = 256
RB = 1024
NEG_INF = np.float32(-np.inf)


def _mm_body(kprev, feat_ref, mult_ref, nm_ref, dego_ref, w_ref,
             out_ref, ro_ref):
    i = pl.program_id(0)
    j = pl.program_id(1)
    dg = dego_ref[...]
    norm = jnp.where(dg > 0, dg ** -0.5, 0.0)
    fn = feat_ref[...] * mult_ref[...]
    out_ref[0] = jnp.dot(fn * norm, w_ref[...],
                         preferred_element_type=jnp.float32)

    @pl.when(j == 0)
    def _():
        @pl.when(i == 0)
        def _():
            ro_ref[...] = jnp.full((2, D), NEG_INF, jnp.float32)
            ro_ref[0:1, :] = jnp.zeros((1, D), jnp.float32)

        ro_ref[0:1, :] += jnp.sum(fn, axis=0, keepdims=True)
        ro_ref[1:2, :] = jnp.maximum(
            ro_ref[1:2, :],
            jnp.max(jnp.where(nm_ref[...] > 0, fn, NEG_INF), axis=0,
                    keepdims=True))

        @pl.when(i == NP // RB - 1)
        def _():
            ro_ref[0:1, :] = ro_ref[0:1, :] / jnp.float32(kprev)


def _scaled_mm(feat, mult, nm, dego, w, kprev):
    return pl.pallas_call(
        functools.partial(_mm_body, kprev),
        grid=(NP // RB, 2),
        in_specs=[
            pl.BlockSpec((RB, D), lambda i, j: (i, 0)),
            pl.BlockSpec((RB, 1), lambda i, j: (i, 0)),
            pl.BlockSpec((RB, 1), lambda i, j: (i, 0)),
            pl.BlockSpec((RB, 1), lambda i, j: (i, 0)),
            pl.BlockSpec((D, 128), lambda i, j: (0, j)),
        ],
        out_specs=[
            pl.BlockSpec((1, RB, 128), lambda i, j: (j, i, 0)),
            pl.BlockSpec((2, D), lambda i, j: (0, 0)),
        ],
        out_shape=[
            jax.ShapeDtypeStruct((2, NP, 128), jnp.float32),
            jax.ShapeDtypeStruct((2, D), jnp.float32),
        ],
        interpret=_INTERPRET,
    )(feat, mult, nm, dego, w)


def _ep_body(agg_ref, di_ref, b_ref, m_ref, wp_ref, do_ref, feat_ref, s_ref):
    agg = jnp.concatenate([agg_ref[0], agg_ref[1]], axis=1)
    dgi = di_ref[...]
    ni = jnp.where(dgi > 0, dgi ** -0.5, 0.0)
    dgo = do_ref[...]
    no = jnp.where(dgo > 0, dgo ** -0.5, 0.0)
    out = jnp.where(m_ref[...] > 0, agg * ni + b_ref[...], 0.0)
    f = jnp.maximum(out, 0.0)
    feat_ref[...] = f
    s_ref[...] = jnp.sum(f * wp_ref[...], axis=1, keepdims=True) * no


def _epilogue(agg, norm_in, b, mask, wp_row, norm_out):
    return pl.pallas_call(
        _ep_body,
        grid=(NP // RB,),
        in_specs=[
            pl.BlockSpec((2, RB, 128), lambda i: (0, i, 0)),
            pl.BlockSpec((RB, 1), lambda i: (i, 0)),
            pl.BlockSpec((1, D), lambda i: (0, 0)),
            pl.BlockSpec((RB, 1), lambda i: (i, 0)),
            pl.BlockSpec((1, D), lambda i: (0, 0)),
            pl.BlockSpec((RB, 1), lambda i: (i, 0)),
        ],
        out_specs=[
            pl.BlockSpec((RB, D), lambda i: (i, 0)),
            pl.BlockSpec((RB, 1), lambda i: (i, 0)),
        ],
        out_shape=[
            jax.ShapeDtypeStruct((NP, D), jnp.float32),
            jax.ShapeDtypeStruct((NP, 1), jnp.float32),
        ],
        interpret=_INTERPRET,
    )(agg, norm_in, b, mask, wp_row, norm_out)


def _top_body(k, aggs_ref, di_ref, bp_ref, m_ref, nm_ref, mult_ref):
    dgi = di_ref[...]
    ni = jnp.where(dgi > 0, dgi ** -0.5, 0.0)
    score = aggs_ref[...] * ni + bp_ref[0, 0]
    sm = jnp.where(m_ref[...] > 0, score, NEG_INF)
    ub = lax.bitcast_convert_type(sm, jnp.uint32)
    top = jnp.uint32(0x80000000)
    u = jnp.where(ub >= top, ~ub, ub | top)

    def bit_body(i, p):
        cand = p | (jnp.uint32(1) << (jnp.uint32(31) - i))
        cnt = jnp.sum((u >= cand).astype(jnp.int32))
        return jnp.where(cnt >= k, cand, p)

    t = lax.fori_loop(0, 32, bit_body, jnp.uint32(0), unroll=True)
    c_gt = jnp.sum((u > t).astype(jnp.int32))
    mrem = k - c_gt
    ties = u == t
    idx = (lax.broadcasted_iota(jnp.int32, (NP // 128, 128), 0) * 128
           + lax.broadcasted_iota(jnp.int32, (NP // 128, 128), 1))

    def tie_body(i, c):
        cand = c + (jnp.int32(1) << (jnp.int32(13) - i))
        cnt = jnp.sum((ties & (idx < cand)).astype(jnp.int32))
        return jnp.where(cnt <= mrem, cand, c)

    cstar = lax.fori_loop(0, 14, tie_body, jnp.int32(0), unroll=True)
    nm = ((u > t) | (ties & (idx < cstar))).astype(jnp.float32)
    nm_ref[...] = nm
    mult_ref[...] = jnp.tanh(sm) * nm


def _sag_topk(aggs80, ni80, bp, mask80, k):
    return pl.pallas_call(
        functools.partial(_top_body, k),
        out_shape=[
            jax.ShapeDtypeStruct((NP // 128, 128), jnp.float32),
            jax.ShapeDtypeStruct((NP // 128, 128), jnp.float32),
        ],
        interpret=_INTERPRET,
    )(aggs80, ni80, bp, mask80)


def _apply_body(k, feat_ref, mult_ref, nm_ref, ro_ref):
    i = pl.program_id(0)
    fn = feat_ref[...] * mult_ref[...]

    @pl.when(i == 0)
    def _():
        ro_ref[...] = jnp.full((2, D), NEG_INF, jnp.float32)
        ro_ref[0:1, :] = jnp.zeros((1, D), jnp.float32)

    ro_ref[0:1, :] += jnp.sum(fn, axis=0, keepdims=True)
    ro_ref[1:2, :] = jnp.maximum(
        ro_ref[1:2, :],
        jnp.max(jnp.where(nm_ref[...] > 0, fn, NEG_INF), axis=0,
                keepdims=True))

    @pl.when(i == NP // RB - 1)
    def _():
        ro_ref[0:1, :] = ro_ref[0:1, :] / jnp.float32(k)


def _apply_pool(feat, mult, nmask, k):
    return pl.pallas_call(
        functools.partial(_apply_body, k),
        grid=(NP // RB,),
        in_specs=[
            pl.BlockSpec((RB, D), lambda i: (i, 0)),
            pl.BlockSpec((RB, 1), lambda i: (i, 0)),
            pl.BlockSpec((RB, 1), lambda i: (i, 0)),
        ],
        out_specs=pl.BlockSpec((2, D), lambda i: (0, 0)),
        out_shape=jax.ShapeDtypeStruct((2, D), jnp.float32),
        interpret=_INTERPRET,
    )(feat, mult, nmask)


def _mlp_body(ro_ref, w1_ref, b1_ref, w2_ref, b2_ref, w3_ref, b3_ref, o_ref):
    r = ro_ref[0] + ro_ref[1] + ro_ref[2]
    avg = r[0:1, :]
    mx = r[1:2, :]
    h = avg @ w1_ref[0:D, :] + mx @ w1_ref[D:2 * D, :] + b1_ref[...]
    h = jnp.maximum(h, 0.0)
    h = jnp.maximum(h @ w2_ref[...] + b2_ref[...], 0.0)
    logits = h @ w3_ref[...] + b3_ref[...]
    m = jnp.max(logits)
    o_ref[...] = logits - (m + jnp.log(jnp.sum(jnp.exp(logits - m))))


def _mlp(ros, w1, b1, w2, b2, w3, b3):
    return pl.pallas_call(
        _mlp_body,
        out_shape=jax.ShapeDtypeStruct((1, 10), jnp.float32),
        interpret=_INTERPRET,
    )(ros, w1, b1, w2, b2, w3, b3)


ECH = 79
ECHA = 158
EPT = ECH * 128
EP = 16 * EPT
NPT = NP // 16
LUT = 1024


def _sc_mesh():
    return plsc.VectorSubcoreMesh(core_axis_name="c", subcore_axis_name="s")


def _zero_vec(ref, n2d):
    z = jnp.zeros((16,), jnp.float32)

    def body(i, _):
        for l in range(8):
            ref[i, pl.ds(l * 16, 16)] = z
        return 0

    lax.fori_loop(0, n2d, body, 0)


def _zero_flat(ref, n):
    z = jnp.zeros((16,), jnp.float32)

    def body(i, _):
        ref[pl.ds(i * 16, 16)] = z
        return 0

    lax.fori_loop(0, n, body, 0)


def _compact_body(src_hbm, dst_hbm, mask_hbm,
                  no_hbm, ni_hbm, srcc_hbm, dstc_hbm, nch_hbm,
                  mask_v, srcb, dstb, srcf, dstf, onesb, ncb, zb,
                  dgb, dego_sp, degi_sp, sem):
    cid = lax.axis_index("c")
    sid = lax.axis_index("s")
    pltpu.sync_copy(mask_hbm, mask_v)
    pltpu.sync_copy(src_hbm.at[sid], srcb)
    pltpu.sync_copy(dst_hbm.at[sid], dstb)
    _zero_flat(zb, NPT // 16)
    _zero_flat(onesb, 8)

    def ofill(i, _):
        onesb[pl.ds(i * 16, 16)] = jnp.ones((16,), jnp.float32)
        return 0

    lax.fori_loop(0, 8, ofill, 0)
    pltpu.sync_copy(zb, dego_sp.at[pl.ds(sid * NPT, NPT)])
    pltpu.sync_copy(zb, degi_sp.at[pl.ds(sid * NPT, NPT)])
    plsc.subcore_barrier()

    def chunk(j, off):
        for l in range(8):
            s16 = srcb[j, pl.ds(l * 16, 16)]
            d16 = dstb[j, pl.ds(l * 16, 16)]
            keep = (plsc.load_gather(mask_v, [s16])
                    * plsc.load_gather(mask_v, [d16])) > 0.0
            plsc.store_compressed(srcf.at[pl.ds(off, 16)], s16, mask=keep)
            plsc.store_compressed(dstf.at[pl.ds(off, 16)], d16, mask=keep)
            off = off + plsc.all_reduce_population_count(keep)[0]
        return off

    off = lax.fori_loop(0, ECH, chunk, 0)
    nch = jnp.maximum((off + 127) // 128, 1)
    end_e = nch * 128
    dv = N + lax.iota(jnp.int32, 16)

    def padloop(w, _):
        base = off + w * 16
        srcf[pl.ds(base, 16)] = dv
        dstf[pl.ds(base, 16)] = dv
        return 0

    lax.fori_loop(0, (end_e - off + 15) // 16, padloop, 0)

    def wrow(j, _):
        pltpu.async_copy(srcf.at[pl.ds(j * 128, 128)],
                         srcc_hbm.at[cid].at[sid].at[j], sem)
        pltpu.async_copy(dstf.at[pl.ds(j * 128, 128)],
                         dstc_hbm.at[cid].at[sid].at[j], sem)
        return 0

    lax.fori_loop(0, nch, wrow, 0)

    def drow(j, _):
        pltpu.make_async_copy(srcf.at[pl.ds(j * 128, 128)],
                              srcc_hbm.at[cid].at[sid].at[j], sem).wait()
        pltpu.make_async_copy(dstf.at[pl.ds(j * 128, 128)],
                              dstc_hbm.at[cid].at[sid].at[j], sem).wait()
        return 0

    lax.fori_loop(0, nch, drow, 0)
    ncb[pl.ds(0, 16)] = jnp.broadcast_to(nch, (16,)).astype(jnp.int32)
    pltpu.sync_copy(ncb, nch_hbm.at[cid].at[sid])

    pltpu.sync_copy(srcc_hbm.at[cid].at[sid], srcb)
    pltpu.sync_copy(dstc_hbm.at[cid].at[sid], dstb)

    def dchunk(j, _):
        c1 = pltpu.async_copy(onesb, dego_sp.at[srcb.at[j]], sem, add=True)
        c2 = pltpu.async_copy(onesb, degi_sp.at[dstb.at[j]], sem, add=True)
        c1.wait()
        c2.wait()
        return 0

    lax.fori_loop(0, nch, dchunk, 0)
    plsc.subcore_barrier()

    base = cid * (NP // 2) + sid * 320
    pltpu.sync_copy(dego_sp.at[pl.ds(base, 320)], dgb)
    pltpu.sync_copy(dgb, no_hbm.at[pl.ds(base, 320)])
    pltpu.sync_copy(degi_sp.at[pl.ds(base, 320)], dgb)
    pltpu.sync_copy(dgb, ni_hbm.at[pl.ds(base, 320)])


def _sc_compact(mask, src3, dst3):
    f = pl.kernel(
        _compact_body,
        out_type=[jax.ShapeDtypeStruct((NP,), jnp.float32),
                  jax.ShapeDtypeStruct((NP,), jnp.float32),
                  jax.ShapeDtypeStruct((2, 16, ECH, 128), jnp.int32),
                  jax.ShapeDtypeStruct((2, 16, ECH, 128), jnp.int32),
                  jax.ShapeDtypeStruct((2, 16, 16), jnp.int32)],
        mesh=_sc_mesh(),
        compiler_params=pltpu.CompilerParams(needs_layout_passes=False),
        scratch_types=[
            pltpu.VMEM((NP,), jnp.float32),
            pltpu.VMEM((ECH, 128), jnp.int32),
            pltpu.VMEM((ECH, 128), jnp.int32),
            pltpu.VMEM((EPT + 32,), jnp.int32),
            pltpu.VMEM((EPT + 32,), jnp.int32),
            pltpu.VMEM((128,), jnp.float32),
            pltpu.VMEM((16,), jnp.int32),
            pltpu.VMEM((NPT,), jnp.float32),
            pltpu.VMEM((320,), jnp.float32),
            pltpu.VMEM_SHARED((NP,), jnp.float32),
            pltpu.VMEM_SHARED((NP,), jnp.float32),
            pltpu.SemaphoreType.DMA,
        ],
    )
    return f(src3, dst3, mask)


NPA = N + 16


def _agg_body(h_hbm, src_hbm, dst_hbm, nch_hbm, out_hbm,
              sring, dring, ncb, gbuf, acc_sp, semi, semg, sems):
    cid = lax.axis_index("c")
    sid = lax.axis_index("s")
    pltpu.sync_copy(nch_hbm.at[cid].at[sid], ncb)
    nch = ncb[pl.ds(0, 16)][0]
    _zero_vec(gbuf.at[0], 128)

    zb0 = sid * (NPA // 16)

    def zloop(t, _):
        pltpu.sync_copy(gbuf.at[0], acc_sp.at[pl.ds(zb0 + t * 128, 128)])
        return 0

    lax.fori_loop(0, 4, zloop, 0)
    pltpu.sync_copy(gbuf.at[0].at[pl.ds(0, NPA // 16 - 512)],
                    acc_sp.at[pl.ds(zb0 + 512, NPA // 16 - 512)])
    plsc.subcore_barrier()

    def si(j):
        pltpu.async_copy(src_hbm.at[cid].at[sid].at[j], sring.at[j % 3],
                         semi)
        pltpu.async_copy(dst_hbm.at[cid].at[sid].at[j], dring.at[j % 3],
                         semi)

    def wi():
        pltpu.make_async_copy(src_hbm.at[cid].at[sid].at[0], sring.at[0],
                              semi).wait()
        pltpu.make_async_copy(dst_hbm.at[cid].at[sid].at[0], dring.at[0],
                              semi).wait()

    def sg(j):
        pltpu.async_copy(h_hbm.at[cid].at[sring.at[j % 3]], gbuf.at[j % 3],
                         semg)

    def wg():
        pltpu.make_async_copy(h_hbm.at[cid].at[sring.at[0]], gbuf.at[0],
                              semg).wait()

    def ss(j):
        pltpu.async_copy(gbuf.at[j % 3], acc_sp.at[dring.at[j % 3]], sems,
                         add=True)

    def ws():
        pltpu.make_async_copy(gbuf.at[0], acc_sp.at[dring.at[0]],
                              sems).wait()

    si(0)
    wi()
    sg(0)

    @pl.when(nch >= 2)
    def _():
        si(1)

    def chunk(j, _):
        wg()

        @pl.when(j + 1 < nch)
        def _():
            wi()

            @pl.when(j >= 2)
            def _():
                ws()

            sg(j + 1)

        ss(j)

        @pl.when(j + 2 < nch)
        def _():
            si(j + 2)

        return 0

    lax.fori_loop(0, nch, chunk, 0)

    @pl.when(nch >= 2)
    def _():
        ws()

    ws()
    plsc.subcore_barrier()

    _zero_vec(gbuf.at[0], 128)
    wb0 = sid * NPT

    @pl.when(sid < 15)
    def _():
        def wloop(t, _):
            rows = wb0 + t * 128
            pltpu.sync_copy(acc_sp.at[pl.ds(rows, 128)], gbuf.at[1])
            pltpu.sync_copy(gbuf.at[1], out_hbm.at[cid].at[pl.ds(rows, 128)])
            return 0

        lax.fori_loop(0, NPT // 128, wloop, 0)

    @pl.when(sid == 15)
    def _():
        def wloop(t, _):
            rows = 15 * NPT + t * 128
            pltpu.sync_copy(acc_sp.at[pl.ds(rows, 128)], gbuf.at[1])
            pltpu.sync_copy(gbuf.at[1], out_hbm.at[cid].at[pl.ds(rows, 128)])
            return 0

        lax.fori_loop(0, 3, wloop, 0)
        pltpu.sync_copy(acc_sp.at[pl.ds(9984, 32)],
                        gbuf.at[1].at[pl.ds(0, 32)])
        pltpu.sync_copy(gbuf.at[1].at[pl.ds(0, 32)],
                        out_hbm.at[cid].at[pl.ds(9984, 32)])
        pltpu.sync_copy(gbuf.at[0].at[pl.ds(0, 96)],
                        out_hbm.at[cid].at[pl.ds(NPA, 96)])
        pltpu.sync_copy(gbuf.at[0], out_hbm.at[cid].at[pl.ds(10112, 128)])


def _sc_aggregate(hsplit, srcc, dstc, nchs):
    f = pl.kernel(
        _agg_body,
        out_type=jax.ShapeDtypeStruct((2, NP, 128), jnp.float32),
        mesh=_sc_mesh(),
        compiler_params=pltpu.CompilerParams(needs_layout_passes=False),
        scratch_types=[
            pltpu.VMEM((3, 128), jnp.int32),
            pltpu.VMEM((3, 128), jnp.int32),
            pltpu.VMEM((16,), jnp.int32),
            pltpu.VMEM((3, 128, 128), jnp.float32),
            pltpu.VMEM_SHARED((NPA, 128), jnp.float32),
            pltpu.SemaphoreType.DMA,
            pltpu.SemaphoreType.DMA,
            pltpu.SemaphoreType.DMA,
        ],
    )
    return f(hsplit, srcc, dstc, nchs)


def _aggs_body(s_hbm, src_hbm, dst_hbm, nch_hbm, out_hbm,
               s_v, srcb, dstb, ncb, sb, zb, agg_sp, sem):
    cid = lax.axis_index("c")
    sid = lax.axis_index("s")
    pltpu.sync_copy(s_hbm, s_v)
    pltpu.sync_copy(src_hbm.at[cid].at[sid], srcb)
    pltpu.sync_copy(dst_hbm.at[cid].at[sid], dstb)
    pltpu.sync_copy(nch_hbm.at[cid].at[sid], ncb)
    nch = ncb[pl.ds(0, 16)][0]
    _zero_flat(zb, NPT // 16)
    pltpu.sync_copy(zb, agg_sp.at[pl.ds(sid * NPT, NPT)])
    plsc.subcore_barrier()

    def chunk(j, _):
        for l in range(8):
            s16 = srcb[j, pl.ds(l * 16, 16)]
            sb[j, pl.ds(l * 16, 16)] = plsc.load_gather(
                s_v, [s16])
        pltpu.async_copy(sb.at[j], agg_sp.at[dstb.at[j]], sem,
                         add=True).wait()
        return 0

    lax.fori_loop(0, nch, chunk, 0)
    plsc.subcore_barrier()
    base = cid * (NP // 2) + sid * 320
    pltpu.sync_copy(agg_sp.at[pl.ds(base, 320)], zb.at[pl.ds(0, 320)])
    pltpu.sync_copy(zb.at[pl.ds(0, 320)], out_hbm.at[pl.ds(base, 320)])


def _sc_aggregate_scalar(s_flat, srcc, dstc, nchs):
    f = pl.kernel(
        _aggs_body,
        out_type=jax.ShapeDtypeStruct((NP,), jnp.float32),
        mesh=_sc_mesh(),
        compiler_params=pltpu.CompilerParams(needs_layout_passes=False),
        scratch_types=[
            pltpu.VMEM((NP,), jnp.float32),
            pltpu.VMEM((ECH, 128), jnp.int32),
            pltpu.VMEM((ECH, 128), jnp.int32),
            pltpu.VMEM((16,), jnp.int32),
            pltpu.VMEM((ECH, 128), jnp.float32),
            pltpu.VMEM((NPT,), jnp.float32),
            pltpu.VMEM_SHARED((NP,), jnp.float32),
            pltpu.SemaphoreType.DMA,
        ],
    )
    return f(s_flat, srcc, dstc, nchs)


def kernel(x, convW0, convb0, poolW0, poolb0, convW1, convb1, poolW1, poolb1,
           convW2, convb2, poolW2, poolb2, lin1W, lin1b, lin2W, lin2b,
           lin3W, lin3b, edge_index):
    pad_idx = N + (jnp.arange(EP - E, dtype=jnp.int32) % (NP - N))
    src_pad = jnp.concatenate([edge_index[0], pad_idx])
    dst_pad = jnp.concatenate([edge_index[1], pad_idx])
    src3 = src_pad.reshape(16, ECH, 128)
    dst3 = dst_pad.reshape(16, ECH, 128)
    feat = jnp.pad(x, ((0, NP - N), (0, 0)))
    mask = (jnp.arange(NP) < N).astype(jnp.float32)
    mult = jnp.ones((NP, 1), jnp.float32)
    nmask = mask[:, None]

    layers = [(convW0, convb0, poolW0, poolb0),
              (convW1, convb1, poolW1, poolb1),
              (convW2, convb2, poolW2, poolb2)]
    ksizes = [5000, 2500, 1250]
    ros = []
    kprev = 1
    for (wc, bc, wp, bp), k in zip(layers, ksizes):
        dego, degi, srcc, dstc, nchs = _sc_compact(mask, src3, dst3)
        hs, ro_prev = _scaled_mm(feat, mult, nmask, dego[:, None], wc,
                                 kprev)
        ros.append(ro_prev)
        agg = _sc_aggregate(hs, srcc, dstc, nchs)
        feat, s_scaled = _epilogue(agg, degi[:, None], bc[None, :],
                                   mask[:, None], wp.reshape(1, D),
                                   dego[:, None])
        aggs = _sc_aggregate_scalar(s_scaled.reshape(NP), srcc, dstc,
                                    nchs)
        nm80, mult80 = _sag_topk(aggs.reshape(NP // 128, 128),
                                 degi.reshape(NP // 128, 128),
                                 bp.reshape(1, 1),
                                 mask.reshape(NP // 128, 128), k)
        nmask = nm80.reshape(NP, 1)
        mult = mult80.reshape(NP, 1)
        mask = nmask[:, 0]
        kprev = k
    ros.append(_apply_pool(feat, mult, nmask, 1250))
    ros = ros[1:]
    return _mlp(jnp.stack(ros), lin1W, lin1b[None, :], lin2W, lin2b[None, :],
                lin3W, lin3b[None, :])


# speedup vs baseline: 29.8257x; 1.0431x over previous
import functools
import math

import numpy as np
import jax
import jax.numpy as jnp
from jax import lax
from jax.experimental import pallas as pl
from jax.experimental.pallas import tpu as pltpu
from jax.experimental.pallas import tpu_sc as plsc

_INTERPRET = False

N = 10000
NP = 10240
E = 160000
D = 256
RB = 1024
NEG_INF = np.float32(-np.inf)


def _mm_body(kprev, feat_ref, mult_ref, nm_ref, dego_ref, w_ref,
             out_ref, ro_ref):
    i = pl.program_id(0)
    j = pl.program_id(1)
    dg = dego_ref[...]
    norm = jnp.where(dg > 0, dg ** -0.5, 0.0)
    fn = feat_ref[...] * mult_ref[...]
    out_ref[0] = jnp.dot(fn * norm, w_ref[...],
                         preferred_element_type=jnp.float32)

    @pl.when(j == 0)
    def _():
        @pl.when(i == 0)
        def _():
            ro_ref[...] = jnp.full((2, D), NEG_INF, jnp.float32)
            ro_ref[0:1, :] = jnp.zeros((1, D), jnp.float32)

        ro_ref[0:1, :] += jnp.sum(fn, axis=0, keepdims=True)
        ro_ref[1:2, :] = jnp.maximum(
            ro_ref[1:2, :],
            jnp.max(jnp.where(nm_ref[...] > 0, fn, NEG_INF), axis=0,
                    keepdims=True))

        @pl.when(i == NP // RB - 1)
        def _():
            ro_ref[0:1, :] = ro_ref[0:1, :] / jnp.float32(kprev)


def _scaled_mm(feat, mult, nm, dego, w, kprev):
    return pl.pallas_call(
        functools.partial(_mm_body, kprev),
        grid=(NP // RB, 2),
        in_specs=[
            pl.BlockSpec((RB, D), lambda i, j: (i, 0)),
            pl.BlockSpec((RB, 1), lambda i, j: (i, 0)),
            pl.BlockSpec((RB, 1), lambda i, j: (i, 0)),
            pl.BlockSpec((RB, 1), lambda i, j: (i, 0)),
            pl.BlockSpec((D, 128), lambda i, j: (0, j)),
        ],
        out_specs=[
            pl.BlockSpec((1, RB, 128), lambda i, j: (j, i, 0)),
            pl.BlockSpec((2, D), lambda i, j: (0, 0)),
        ],
        out_shape=[
            jax.ShapeDtypeStruct((2, NP, 128), jnp.float32),
            jax.ShapeDtypeStruct((2, D), jnp.float32),
        ],
        interpret=_INTERPRET,
    )(feat, mult, nm, dego, w)


def _ep_body(agg_ref, di_ref, b_ref, m_ref, wp_ref, do_ref, feat_ref, s_ref):
    agg = jnp.concatenate([agg_ref[0], agg_ref[1]], axis=1)
    dgi = di_ref[...]
    ni = jnp.where(dgi > 0, dgi ** -0.5, 0.0)
    dgo = do_ref[...]
    no = jnp.where(dgo > 0, dgo ** -0.5, 0.0)
    out = jnp.where(m_ref[...] > 0, agg * ni + b_ref[...], 0.0)
    f = jnp.maximum(out, 0.0)
    feat_ref[...] = f
    s_ref[...] = jnp.sum(f * wp_ref[...], axis=1, keepdims=True) * no


def _epilogue(agg, norm_in, b, mask, wp_row, norm_out):
    return pl.pallas_call(
        _ep_body,
        grid=(NP // RB,),
        in_specs=[
            pl.BlockSpec((2, RB, 128), lambda i: (0, i, 0)),
            pl.BlockSpec((RB, 1), lambda i: (i, 0)),
            pl.BlockSpec((1, D), lambda i: (0, 0)),
            pl.BlockSpec((RB, 1), lambda i: (i, 0)),
            pl.BlockSpec((1, D), lambda i: (0, 0)),
            pl.BlockSpec((RB, 1), lambda i: (i, 0)),
        ],
        out_specs=[
            pl.BlockSpec((RB, D), lambda i: (i, 0)),
            pl.BlockSpec((RB, 1), lambda i: (i, 0)),
        ],
        out_shape=[
            jax.ShapeDtypeStruct((NP, D), jnp.float32),
            jax.ShapeDtypeStruct((NP, 1), jnp.float32),
        ],
        interpret=_INTERPRET,
    )(agg, norm_in, b, mask, wp_row, norm_out)


def _top_body(k, aggs_ref, di_ref, bp_ref, m_ref, nm_ref, mult_ref):
    dgi = di_ref[...]
    ni = jnp.where(dgi > 0, dgi ** -0.5, 0.0)
    score = aggs_ref[...] * ni + bp_ref[0, 0]
    sm = jnp.where(m_ref[...] > 0, score, NEG_INF)
    ub = lax.bitcast_convert_type(sm, jnp.uint32)
    top = jnp.uint32(0x80000000)
    u = jnp.where(ub >= top, ~ub, ub | top)

    def bit_body(i, p):
        cand = p | (jnp.uint32(1) << (jnp.uint32(31) - i))
        cnt = jnp.sum((u >= cand).astype(jnp.int32))
        return jnp.where(cnt >= k, cand, p)

    t = lax.fori_loop(0, 32, bit_body, jnp.uint32(0), unroll=True)
    c_gt = jnp.sum((u > t).astype(jnp.int32))
    mrem = k - c_gt
    ties = u == t
    idx = (lax.broadcasted_iota(jnp.int32, (NP // 128, 128), 0) * 128
           + lax.broadcasted_iota(jnp.int32, (NP // 128, 128), 1))

    def tie_body(i, c):
        cand = c + (jnp.int32(1) << (jnp.int32(13) - i))
        cnt = jnp.sum((ties & (idx < cand)).astype(jnp.int32))
        return jnp.where(cnt <= mrem, cand, c)

    cstar = lax.fori_loop(0, 14, tie_body, jnp.int32(0), unroll=True)
    nm = ((u > t) | (ties & (idx < cstar))).astype(jnp.float32)
    nm_ref[...] = nm
    mult_ref[...] = jnp.tanh(sm) * nm


def _sag_topk(aggs80, ni80, bp, mask80, k):
    return pl.pallas_call(
        functools.partial(_top_body, k),
        out_shape=[
            jax.ShapeDtypeStruct((NP // 128, 128), jnp.float32),
            jax.ShapeDtypeStruct((NP // 128, 128), jnp.float32),
        ],
        interpret=_INTERPRET,
    )(aggs80, ni80, bp, mask80)


def _apply_body(k, feat_ref, mult_ref, nm_ref, ro_ref):
    i = pl.program_id(0)
    fn = feat_ref[...] * mult_ref[...]

    @pl.when(i == 0)
    def _():
        ro_ref[...] = jnp.full((2, D), NEG_INF, jnp.float32)
        ro_ref[0:1, :] = jnp.zeros((1, D), jnp.float32)

    ro_ref[0:1, :] += jnp.sum(fn, axis=0, keepdims=True)
    ro_ref[1:2, :] = jnp.maximum(
        ro_ref[1:2, :],
        jnp.max(jnp.where(nm_ref[...] > 0, fn, NEG_INF), axis=0,
                keepdims=True))

    @pl.when(i == NP // RB - 1)
    def _():
        ro_ref[0:1, :] = ro_ref[0:1, :] / jnp.float32(k)


def _apply_pool(feat, mult, nmask, k):
    return pl.pallas_call(
        functools.partial(_apply_body, k),
        grid=(NP // RB,),
        in_specs=[
            pl.BlockSpec((RB, D), lambda i: (i, 0)),
            pl.BlockSpec((RB, 1), lambda i: (i, 0)),
            pl.BlockSpec((RB, 1), lambda i: (i, 0)),
        ],
        out_specs=pl.BlockSpec((2, D), lambda i: (0, 0)),
        out_shape=jax.ShapeDtypeStruct((2, D), jnp.float32),
        interpret=_INTERPRET,
    )(feat, mult, nmask)


def _mlp_body(ro_ref, w1_ref, b1_ref, w2_ref, b2_ref, w3_ref, b3_ref, o_ref):
    r = ro_ref[0] + ro_ref[1] + ro_ref[2]
    avg = r[0:1, :]
    mx = r[1:2, :]
    h = avg @ w1_ref[0:D, :] + mx @ w1_ref[D:2 * D, :] + b1_ref[...]
    h = jnp.maximum(h, 0.0)
    h = jnp.maximum(h @ w2_ref[...] + b2_ref[...], 0.0)
    logits = h @ w3_ref[...] + b3_ref[...]
    m = jnp.max(logits)
    o_ref[...] = logits - (m + jnp.log(jnp.sum(jnp.exp(logits - m))))


def _mlp(ros, w1, b1, w2, b2, w3, b3):
    return pl.pallas_call(
        _mlp_body,
        out_shape=jax.ShapeDtypeStruct((1, 10), jnp.float32),
        interpret=_INTERPRET,
    )(ros, w1, b1, w2, b2, w3, b3)


ECH = 79
ECHA = 158
EPT = ECH * 128
EP = 16 * EPT
NPT = NP // 16
LUT = 1024


def _sc_mesh():
    return plsc.VectorSubcoreMesh(core_axis_name="c", subcore_axis_name="s")


def _zero_vec(ref, n2d):
    z = jnp.zeros((16,), jnp.float32)

    def body(i, _):
        for l in range(8):
            ref[i, pl.ds(l * 16, 16)] = z
        return 0

    lax.fori_loop(0, n2d, body, 0)


def _zero_flat(ref, n):
    z = jnp.zeros((16,), jnp.float32)

    def body(i, _):
        ref[pl.ds(i * 16, 16)] = z
        return 0

    lax.fori_loop(0, n, body, 0)


def _compact_body(src_hbm, dst_hbm, mask_hbm,
                  no_hbm, ni_hbm, srcc_hbm, dstc_hbm, nch_hbm,
                  mask_v, srcb, dstb, srcf, dstf, onesb, ncb, zb,
                  dgb, dego_sp, degi_sp, sem):
    cid = lax.axis_index("c")
    sid = lax.axis_index("s")
    pltpu.sync_copy(mask_hbm, mask_v)
    pltpu.sync_copy(src_hbm.at[sid], srcb)
    pltpu.sync_copy(dst_hbm.at[sid], dstb)
    _zero_flat(zb, NPT // 16)
    _zero_flat(onesb, 8)

    def ofill(i, _):
        onesb[pl.ds(i * 16, 16)] = jnp.ones((16,), jnp.float32)
        return 0

    lax.fori_loop(0, 8, ofill, 0)
    pltpu.sync_copy(zb, dego_sp.at[pl.ds(sid * NPT, NPT)])
    pltpu.sync_copy(zb, degi_sp.at[pl.ds(sid * NPT, NPT)])
    plsc.subcore_barrier()

    def chunk(j, off):
        for l in range(8):
            s16 = srcb[j, pl.ds(l * 16, 16)]
            d16 = dstb[j, pl.ds(l * 16, 16)]
            keep = (plsc.load_gather(mask_v, [s16])
                    * plsc.load_gather(mask_v, [d16])) > 0.0
            plsc.store_compressed(srcf.at[pl.ds(off, 16)], s16, mask=keep)
            plsc.store_compressed(dstf.at[pl.ds(off, 16)], d16, mask=keep)
            off = off + plsc.all_reduce_population_count(keep)[0]
        return off

    off = lax.fori_loop(0, ECH, chunk, 0)
    nch = jnp.maximum((off + 127) // 128, 1)
    end_e = nch * 128
    dv = N + lax.iota(jnp.int32, 16)

    def padloop(w, _):
        base = off + w * 16
        srcf[pl.ds(base, 16)] = dv
        dstf[pl.ds(base, 16)] = dv
        return 0

    lax.fori_loop(0, (end_e - off + 15) // 16, padloop, 0)

    def wrow(j, _):
        pltpu.async_copy(srcf.at[pl.ds(j * 128, 128)],
                         srcc_hbm.at[cid].at[sid].at[j], sem)
        pltpu.async_copy(dstf.at[pl.ds(j * 128, 128)],
                         dstc_hbm.at[cid].at[sid].at[j], sem)
        return 0

    lax.fori_loop(0, nch, wrow, 0)

    def drow(j, _):
        pltpu.make_async_copy(srcf.at[pl.ds(j * 128, 128)],
                              srcc_hbm.at[cid].at[sid].at[j], sem).wait()
        pltpu.make_async_copy(dstf.at[pl.ds(j * 128, 128)],
                              dstc_hbm.at[cid].at[sid].at[j], sem).wait()
        return 0

    lax.fori_loop(0, nch, drow, 0)
    ncb[pl.ds(0, 16)] = jnp.broadcast_to(nch, (16,)).astype(jnp.int32)
    pltpu.sync_copy(ncb, nch_hbm.at[cid].at[sid])

    pltpu.sync_copy(srcc_hbm.at[cid].at[sid], srcb)
    pltpu.sync_copy(dstc_hbm.at[cid].at[sid], dstb)

    def dchunk(j, _):
        pltpu.async_copy(onesb, dego_sp.at[srcb.at[j]], sem, add=True)
        pltpu.async_copy(onesb, degi_sp.at[dstb.at[j]], sem, add=True)
        return 0

    lax.fori_loop(0, nch, dchunk, 0)

    def ddrain(j, _):
        pltpu.make_async_copy(onesb, dego_sp.at[srcb.at[0]], sem).wait()
        pltpu.make_async_copy(onesb, degi_sp.at[dstb.at[0]], sem).wait()
        return 0

    lax.fori_loop(0, nch, ddrain, 0)
    plsc.subcore_barrier()

    base = cid * (NP // 2) + sid * 320
    pltpu.sync_copy(dego_sp.at[pl.ds(base, 320)], dgb)
    pltpu.sync_copy(dgb, no_hbm.at[pl.ds(base, 320)])
    pltpu.sync_copy(degi_sp.at[pl.ds(base, 320)], dgb)
    pltpu.sync_copy(dgb, ni_hbm.at[pl.ds(base, 320)])


def _sc_compact(mask, src3, dst3):
    f = pl.kernel(
        _compact_body,
        out_type=[jax.ShapeDtypeStruct((NP,), jnp.float32),
                  jax.ShapeDtypeStruct((NP,), jnp.float32),
                  jax.ShapeDtypeStruct((2, 16, ECH, 128), jnp.int32),
                  jax.ShapeDtypeStruct((2, 16, ECH, 128), jnp.int32),
                  jax.ShapeDtypeStruct((2, 16, 16), jnp.int32)],
        mesh=_sc_mesh(),
        compiler_params=pltpu.CompilerParams(needs_layout_passes=False),
        scratch_types=[
            pltpu.VMEM((NP,), jnp.float32),
            pltpu.VMEM((ECH, 128), jnp.int32),
            pltpu.VMEM((ECH, 128), jnp.int32),
            pltpu.VMEM((EPT + 32,), jnp.int32),
            pltpu.VMEM((EPT + 32,), jnp.int32),
            pltpu.VMEM((128,), jnp.float32),
            pltpu.VMEM((16,), jnp.int32),
            pltpu.VMEM((NPT,), jnp.float32),
            pltpu.VMEM((320,), jnp.float32),
            pltpu.VMEM_SHARED((NP,), jnp.float32),
            pltpu.VMEM_SHARED((NP,), jnp.float32),
            pltpu.SemaphoreType.DMA,
        ],
    )
    return f(src3, dst3, mask)


NPA = N + 16


def _agg_body(h_hbm, src_hbm, dst_hbm, nch_hbm, out_hbm,
              sring, dring, ncb, gbuf, acc_sp, semi, semg, sems, semz):
    cid = lax.axis_index("c")
    sid = lax.axis_index("s")
    pltpu.sync_copy(nch_hbm.at[cid].at[sid], ncb)
    nch = ncb[pl.ds(0, 16)][0]
    _zero_vec(gbuf.at[0], 128)

    zb0 = sid * (NPA // 16)

    def zloop(t, _):
        pltpu.async_copy(gbuf.at[0], acc_sp.at[pl.ds(zb0 + t * 128, 128)],
                         semz)
        return 0

    lax.fori_loop(0, 4, zloop, 0)
    pltpu.async_copy(gbuf.at[0].at[pl.ds(0, NPA // 16 - 512)],
                     acc_sp.at[pl.ds(zb0 + 512, NPA // 16 - 512)], semz)

    def si(j):
        pltpu.async_copy(src_hbm.at[cid].at[sid].at[j], sring.at[j % 3],
                         semi)
        pltpu.async_copy(dst_hbm.at[cid].at[sid].at[j], dring.at[j % 3],
                         semi)

    def wi():
        pltpu.make_async_copy(src_hbm.at[cid].at[sid].at[0], sring.at[0],
                              semi).wait()
        pltpu.make_async_copy(dst_hbm.at[cid].at[sid].at[0], dring.at[0],
                              semi).wait()

    def sg(j):
        pltpu.async_copy(h_hbm.at[cid].at[sring.at[j % 3]], gbuf.at[j % 3],
                         semg)

    def wg():
        pltpu.make_async_copy(h_hbm.at[cid].at[sring.at[0]], gbuf.at[0],
                              semg).wait()

    def ss(j):
        pltpu.async_copy(gbuf.at[j % 3], acc_sp.at[dring.at[j % 3]], sems,
                         add=True)

    def ws():
        pltpu.make_async_copy(gbuf.at[0], acc_sp.at[dring.at[0]],
                              sems).wait()

    si(0)
    wi()
    sg(0)

    @pl.when(nch >= 2)
    def _():
        si(1)

    def zdrain(t, _):
        pltpu.make_async_copy(gbuf.at[0],
                              acc_sp.at[pl.ds(zb0, 128)], semz).wait()
        return 0

    lax.fori_loop(0, 4, zdrain, 0)
    pltpu.make_async_copy(gbuf.at[0].at[pl.ds(0, NPA // 16 - 512)],
                          acc_sp.at[pl.ds(zb0 + 512, NPA // 16 - 512)],
                          semz).wait()
    plsc.subcore_barrier()

    def chunk(j, _):
        wg()

        @pl.when(j + 1 < nch)
        def _():
            wi()

            @pl.when(j >= 2)
            def _():
                ws()

            sg(j + 1)

        ss(j)

        @pl.when(j + 2 < nch)
        def _():
            si(j + 2)

        return 0

    lax.fori_loop(0, nch, chunk, 0)

    @pl.when(nch >= 2)
    def _():
        ws()

    ws()
    plsc.subcore_barrier()

    _zero_vec(gbuf.at[0], 128)
    wb0 = sid * NPT

    @pl.when(sid < 15)
    def _():
        def wloop(t, _):
            rows = wb0 + t * 128
            buf = gbuf.at[1 + t % 2]
            pltpu.sync_copy(acc_sp.at[pl.ds(rows, 128)], buf)
            pltpu.async_copy(buf, out_hbm.at[cid].at[pl.ds(rows, 128)],
                             semg)
            return 0

        lax.fori_loop(0, NPT // 128, wloop, 0)

        def wdrain(t, _):
            pltpu.make_async_copy(gbuf.at[1],
                                  out_hbm.at[cid].at[pl.ds(wb0, 128)],
                                  semg).wait()
            return 0

        lax.fori_loop(0, NPT // 128, wdrain, 0)

    @pl.when(sid == 15)
    def _():
        def wloop(t, _):
            rows = 15 * NPT + t * 128
            pltpu.sync_copy(acc_sp.at[pl.ds(rows, 128)], gbuf.at[1])
            pltpu.sync_copy(gbuf.at[1], out_hbm.at[cid].at[pl.ds(rows, 128)])
            return 0

        lax.fori_loop(0, 3, wloop, 0)
        pltpu.sync_copy(acc_sp.at[pl.ds(9984, 32)],
                        gbuf.at[1].at[pl.ds(0, 32)])
        pltpu.sync_copy(gbuf.at[1].at[pl.ds(0, 32)],
                        out_hbm.at[cid].at[pl.ds(9984, 32)])
        pltpu.sync_copy(gbuf.at[0].at[pl.ds(0, 96)],
                        out_hbm.at[cid].at[pl.ds(NPA, 96)])
        pltpu.sync_copy(gbuf.at[0], out_hbm.at[cid].at[pl.ds(10112, 128)])


def _sc_aggregate(hsplit, srcc, dstc, nchs):
    f = pl.kernel(
        _agg_body,
        out_type=jax.ShapeDtypeStruct((2, NP, 128), jnp.float32),
        mesh=_sc_mesh(),
        compiler_params=pltpu.CompilerParams(needs_layout_passes=False),
        scratch_types=[
            pltpu.VMEM((3, 128), jnp.int32),
            pltpu.VMEM((3, 128), jnp.int32),
            pltpu.VMEM((16,), jnp.int32),
            pltpu.VMEM((3, 128, 128), jnp.float32),
            pltpu.VMEM_SHARED((NPA, 128), jnp.float32),
            pltpu.SemaphoreType.DMA,
            pltpu.SemaphoreType.DMA,
            pltpu.SemaphoreType.DMA,
            pltpu.SemaphoreType.DMA,
        ],
    )
    return f(hsplit, srcc, dstc, nchs)


def _aggs_body(s_hbm, src_hbm, dst_hbm, nch_hbm, out_hbm,
               s_v, srcb, dstb, ncb, sb, zb, agg_sp, sem):
    cid = lax.axis_index("c")
    sid = lax.axis_index("s")
    pltpu.sync_copy(s_hbm, s_v)
    pltpu.sync_copy(src_hbm.at[cid].at[sid], srcb)
    pltpu.sync_copy(dst_hbm.at[cid].at[sid], dstb)
    pltpu.sync_copy(nch_hbm.at[cid].at[sid], ncb)
    nch = ncb[pl.ds(0, 16)][0]
    _zero_flat(zb, NPT // 16)
    pltpu.sync_copy(zb, agg_sp.at[pl.ds(sid * NPT, NPT)])
    plsc.subcore_barrier()

    def chunk(j, _):
        for l in range(8):
            s16 = srcb[j, pl.ds(l * 16, 16)]
            sb[j, pl.ds(l * 16, 16)] = plsc.load_gather(
                s_v, [s16])
        pltpu.async_copy(sb.at[j], agg_sp.at[dstb.at[j]], sem, add=True)
        return 0

    lax.fori_loop(0, nch, chunk, 0)

    def sdrain(j, _):
        pltpu.make_async_copy(sb.at[0], agg_sp.at[dstb.at[0]], sem).wait()
        return 0

    lax.fori_loop(0, nch, sdrain, 0)
    plsc.subcore_barrier()
    base = cid * (NP // 2) + sid * 320
    pltpu.sync_copy(agg_sp.at[pl.ds(base, 320)], zb.at[pl.ds(0, 320)])
    pltpu.sync_copy(zb.at[pl.ds(0, 320)], out_hbm.at[pl.ds(base, 320)])


def _sc_aggregate_scalar(s_flat, srcc, dstc, nchs):
    f = pl.kernel(
        _aggs_body,
        out_type=jax.ShapeDtypeStruct((NP,), jnp.float32),
        mesh=_sc_mesh(),
        compiler_params=pltpu.CompilerParams(needs_layout_passes=False),
        scratch_types=[
            pltpu.VMEM((NP,), jnp.float32),
            pltpu.VMEM((ECH, 128), jnp.int32),
            pltpu.VMEM((ECH, 128), jnp.int32),
            pltpu.VMEM((16,), jnp.int32),
            pltpu.VMEM((ECH, 128), jnp.float32),
            pltpu.VMEM((NPT,), jnp.float32),
            pltpu.VMEM_SHARED((NP,), jnp.float32),
            pltpu.SemaphoreType.DMA,
        ],
    )
    return f(s_flat, srcc, dstc, nchs)


def kernel(x, convW0, convb0, poolW0, poolb0, convW1, convb1, poolW1, poolb1,
           convW2, convb2, poolW2, poolb2, lin1W, lin1b, lin2W, lin2b,
           lin3W, lin3b, edge_index):
    pad_idx = N + (jnp.arange(EP - E, dtype=jnp.int32) % (NP - N))
    src_pad = jnp.concatenate([edge_index[0], pad_idx])
    dst_pad = jnp.concatenate([edge_index[1], pad_idx])
    src3 = src_pad.reshape(16, ECH, 128)
    dst3 = dst_pad.reshape(16, ECH, 128)
    feat = jnp.pad(x, ((0, NP - N), (0, 0)))
    mask = (jnp.arange(NP) < N).astype(jnp.float32)
    mult = jnp.ones((NP, 1), jnp.float32)
    nmask = mask[:, None]

    layers = [(convW0, convb0, poolW0, poolb0),
              (convW1, convb1, poolW1, poolb1),
              (convW2, convb2, poolW2, poolb2)]
    ksizes = [5000, 2500, 1250]
    ros = []
    kprev = 1
    for (wc, bc, wp, bp), k in zip(layers, ksizes):
        dego, degi, srcc, dstc, nchs = _sc_compact(mask, src3, dst3)
        hs, ro_prev = _scaled_mm(feat, mult, nmask, dego[:, None], wc,
                                 kprev)
        ros.append(ro_prev)
        agg = _sc_aggregate(hs, srcc, dstc, nchs)
        feat, s_scaled = _epilogue(agg, degi[:, None], bc[None, :],
                                   mask[:, None], wp.reshape(1, D),
                                   dego[:, None])
        aggs = _sc_aggregate_scalar(s_scaled.reshape(NP), srcc, dstc,
                                    nchs)
        nm80, mult80 = _sag_topk(aggs.reshape(NP // 128, 128),
                                 degi.reshape(NP // 128, 128),
                                 bp.reshape(1, 1),
                                 mask.reshape(NP // 128, 128), k)
        nmask = nm80.reshape(NP, 1)
        mult = mult80.reshape(NP, 1)
        mask = nmask[:, 0]
        kprev = k
    ros.append(_apply_pool(feat, mult, nmask, 1250))
    ros = ros[1:]
    return _mlp(jnp.stack(ros), lin1W, lin1b[None, :], lin2W, lin2b[None, :],
                lin3W, lin3b[None, :])


# speedup vs baseline: 29.8351x; 1.0003x over previous
import functools

import numpy as np
import jax
import jax.numpy as jnp
from jax import lax
from jax.experimental import pallas as pl
from jax.experimental.pallas import tpu as pltpu
from jax.experimental.pallas import tpu_sc as plsc

N = 10000
NP = 10240
E = 160000
D = 256
RB = 1024
NEG_INF = np.float32(-np.inf)


def _mm_body(kprev, feat_ref, mult_ref, nm_ref, dego_ref, w_ref,
             out_ref, ro_ref):
    i = pl.program_id(0)
    j = pl.program_id(1)
    dg = dego_ref[...]
    norm = jnp.where(dg > 0, dg ** -0.5, 0.0)
    fn = feat_ref[...] * mult_ref[...]
    out_ref[0] = jnp.dot(fn * norm, w_ref[...],
                         preferred_element_type=jnp.float32)

    @pl.when(j == 0)
    def _():
        @pl.when(i == 0)
        def _():
            ro_ref[...] = jnp.full((2, D), NEG_INF, jnp.float32)
            ro_ref[0:1, :] = jnp.zeros((1, D), jnp.float32)

        ro_ref[0:1, :] += jnp.sum(fn, axis=0, keepdims=True)
        ro_ref[1:2, :] = jnp.maximum(
            ro_ref[1:2, :],
            jnp.max(jnp.where(nm_ref[...] > 0, fn, NEG_INF), axis=0,
                    keepdims=True))

        @pl.when(i == NP // RB - 1)
        def _():
            ro_ref[0:1, :] = ro_ref[0:1, :] / jnp.float32(kprev)


def _scaled_mm(feat, mult, nm, dego, w, kprev):
    return pl.pallas_call(
        functools.partial(_mm_body, kprev),
        grid=(NP // RB, 2),
        in_specs=[
            pl.BlockSpec((RB, D), lambda i, j: (i, 0)),
            pl.BlockSpec((RB, 1), lambda i, j: (i, 0)),
            pl.BlockSpec((RB, 1), lambda i, j: (i, 0)),
            pl.BlockSpec((RB, 1), lambda i, j: (i, 0)),
            pl.BlockSpec((D, 128), lambda i, j: (0, j)),
        ],
        out_specs=[
            pl.BlockSpec((1, RB, 128), lambda i, j: (j, i, 0)),
            pl.BlockSpec((2, D), lambda i, j: (0, 0)),
        ],
        out_shape=[
            jax.ShapeDtypeStruct((2, NP, 128), jnp.float32),
            jax.ShapeDtypeStruct((2, D), jnp.float32),
        ],
    )(feat, mult, nm, dego, w)


def _ep_body(agg_ref, di_ref, b_ref, m_ref, wp_ref, do_ref, feat_ref, s_ref):
    agg = jnp.concatenate([agg_ref[0], agg_ref[1]], axis=1)
    dgi = di_ref[...]
    ni = jnp.where(dgi > 0, dgi ** -0.5, 0.0)
    dgo = do_ref[...]
    no = jnp.where(dgo > 0, dgo ** -0.5, 0.0)
    out = jnp.where(m_ref[...] > 0, agg * ni + b_ref[...], 0.0)
    f = jnp.maximum(out, 0.0)
    feat_ref[...] = f
    s_ref[...] = jnp.sum(f * wp_ref[...], axis=1, keepdims=True) * no


def _epilogue(agg, norm_in, b, mask, wp_row, norm_out):
    return pl.pallas_call(
        _ep_body,
        grid=(NP // RB,),
        in_specs=[
            pl.BlockSpec((2, RB, 128), lambda i: (0, i, 0)),
            pl.BlockSpec((RB, 1), lambda i: (i, 0)),
            pl.BlockSpec((1, D), lambda i: (0, 0)),
            pl.BlockSpec((RB, 1), lambda i: (i, 0)),
            pl.BlockSpec((1, D), lambda i: (0, 0)),
            pl.BlockSpec((RB, 1), lambda i: (i, 0)),
        ],
        out_specs=[
            pl.BlockSpec((RB, D), lambda i: (i, 0)),
            pl.BlockSpec((RB, 1), lambda i: (i, 0)),
        ],
        out_shape=[
            jax.ShapeDtypeStruct((NP, D), jnp.float32),
            jax.ShapeDtypeStruct((NP, 1), jnp.float32),
        ],
    )(agg, norm_in, b, mask, wp_row, norm_out)


def _top_body(k, aggs_ref, di_ref, bp_ref, m_ref, nm_ref, mult_ref):
    dgi = di_ref[...]
    ni = jnp.where(dgi > 0, dgi ** -0.5, 0.0)
    score = aggs_ref[...] * ni + bp_ref[0, 0]
    sm = jnp.where(m_ref[...] > 0, score, NEG_INF)
    ub = lax.bitcast_convert_type(sm, jnp.uint32)
    top = jnp.uint32(0x80000000)
    u = jnp.where(ub >= top, ~ub, ub | top)

    def bit_body(i, p):
        cand = p | (jnp.uint32(1) << (jnp.uint32(31) - i))
        cnt = jnp.sum((u >= cand).astype(jnp.int32))
        return jnp.where(cnt >= k, cand, p)

    t = lax.fori_loop(0, 32, bit_body, jnp.uint32(0), unroll=True)
    c_gt = jnp.sum((u > t).astype(jnp.int32))
    mrem = k - c_gt
    ties = u == t
    idx = (lax.broadcasted_iota(jnp.int32, (NP // 128, 128), 0) * 128
           + lax.broadcasted_iota(jnp.int32, (NP // 128, 128), 1))

    def tie_body(i, c):
        cand = c + (jnp.int32(1) << (jnp.int32(13) - i))
        cnt = jnp.sum((ties & (idx < cand)).astype(jnp.int32))
        return jnp.where(cnt <= mrem, cand, c)

    cstar = lax.fori_loop(0, 14, tie_body, jnp.int32(0), unroll=True)
    nm = ((u > t) | (ties & (idx < cstar))).astype(jnp.float32)
    nm_ref[...] = nm
    mult_ref[...] = jnp.tanh(sm) * nm


def _sag_topk(aggs80, ni80, bp, mask80, k):
    return pl.pallas_call(
        functools.partial(_top_body, k),
        out_shape=[
            jax.ShapeDtypeStruct((NP // 128, 128), jnp.float32),
            jax.ShapeDtypeStruct((NP // 128, 128), jnp.float32),
        ],
    )(aggs80, ni80, bp, mask80)


def _apply_body(k, feat_ref, mult_ref, nm_ref, ro_ref):
    i = pl.program_id(0)
    fn = feat_ref[...] * mult_ref[...]

    @pl.when(i == 0)
    def _():
        ro_ref[...] = jnp.full((2, D), NEG_INF, jnp.float32)
        ro_ref[0:1, :] = jnp.zeros((1, D), jnp.float32)

    ro_ref[0:1, :] += jnp.sum(fn, axis=0, keepdims=True)
    ro_ref[1:2, :] = jnp.maximum(
        ro_ref[1:2, :],
        jnp.max(jnp.where(nm_ref[...] > 0, fn, NEG_INF), axis=0,
                keepdims=True))

    @pl.when(i == NP // RB - 1)
    def _():
        ro_ref[0:1, :] = ro_ref[0:1, :] / jnp.float32(k)


def _apply_pool(feat, mult, nmask, k):
    return pl.pallas_call(
        functools.partial(_apply_body, k),
        grid=(NP // RB,),
        in_specs=[
            pl.BlockSpec((RB, D), lambda i: (i, 0)),
            pl.BlockSpec((RB, 1), lambda i: (i, 0)),
            pl.BlockSpec((RB, 1), lambda i: (i, 0)),
        ],
        out_specs=pl.BlockSpec((2, D), lambda i: (0, 0)),
        out_shape=jax.ShapeDtypeStruct((2, D), jnp.float32),
    )(feat, mult, nmask)


def _mlp_body(ro_ref, w1_ref, b1_ref, w2_ref, b2_ref, w3_ref, b3_ref, o_ref):
    r = ro_ref[0] + ro_ref[1] + ro_ref[2]
    avg = r[0:1, :]
    mx = r[1:2, :]
    h = avg @ w1_ref[0:D, :] + mx @ w1_ref[D:2 * D, :] + b1_ref[...]
    h = jnp.maximum(h, 0.0)
    h = jnp.maximum(h @ w2_ref[...] + b2_ref[...], 0.0)
    logits = h @ w3_ref[...] + b3_ref[...]
    m = jnp.max(logits)
    o_ref[...] = logits - (m + jnp.log(jnp.sum(jnp.exp(logits - m))))


def _mlp(ros, w1, b1, w2, b2, w3, b3):
    return pl.pallas_call(
        _mlp_body,
        out_shape=jax.ShapeDtypeStruct((1, 10), jnp.float32),
    )(ros, w1, b1, w2, b2, w3, b3)


ECH = 79
ECHA = 158
EPT = ECH * 128
EP = 16 * EPT
NPT = NP // 16


def _sc_mesh():
    return plsc.VectorSubcoreMesh(core_axis_name="c", subcore_axis_name="s")


def _zero_vec(ref, n2d):
    z = jnp.zeros((16,), jnp.float32)

    def body(i, _):
        for l in range(8):
            ref[i, pl.ds(l * 16, 16)] = z
        return 0

    lax.fori_loop(0, n2d, body, 0)


def _zero_flat(ref, n):
    z = jnp.zeros((16,), jnp.float32)

    def body(i, _):
        ref[pl.ds(i * 16, 16)] = z
        return 0

    lax.fori_loop(0, n, body, 0)


def _compact_body(src_hbm, dst_hbm, mask_hbm,
                  no_hbm, ni_hbm, srcc_hbm, dstc_hbm, nch_hbm,
                  mask_v, srcb, dstb, srcf, dstf, onesb, ncb, zb,
                  dgb, dego_sp, degi_sp, sem):
    cid = lax.axis_index("c")
    sid = lax.axis_index("s")
    pltpu.sync_copy(mask_hbm, mask_v)
    pltpu.sync_copy(src_hbm.at[sid], srcb)
    pltpu.sync_copy(dst_hbm.at[sid], dstb)
    _zero_flat(zb, NPT // 16)

    def ofill(i, _):
        onesb[pl.ds(i * 16, 16)] = jnp.ones((16,), jnp.float32)
        return 0

    lax.fori_loop(0, 8, ofill, 0)
    pltpu.sync_copy(zb, dego_sp.at[pl.ds(sid * NPT, NPT)])
    pltpu.sync_copy(zb, degi_sp.at[pl.ds(sid * NPT, NPT)])
    plsc.subcore_barrier()

    def chunk(j, off):
        for l in range(8):
            s16 = srcb[j, pl.ds(l * 16, 16)]
            d16 = dstb[j, pl.ds(l * 16, 16)]
            keep = (plsc.load_gather(mask_v, [s16])
                    * plsc.load_gather(mask_v, [d16])) > 0.0
            plsc.store_compressed(srcf.at[pl.ds(off, 16)], s16, mask=keep)
            plsc.store_compressed(dstf.at[pl.ds(off, 16)], d16, mask=keep)
            off = off + plsc.all_reduce_population_count(keep)[0]
        return off

    off = lax.fori_loop(0, ECH, chunk, 0)
    nch = jnp.maximum((off + 127) // 128, 1)
    end_e = nch * 128
    dv = N + lax.iota(jnp.int32, 16)

    def padloop(w, _):
        base = off + w * 16
        srcf[pl.ds(base, 16)] = dv
        dstf[pl.ds(base, 16)] = dv
        return 0

    lax.fori_loop(0, (end_e - off + 15) // 16, padloop, 0)

    def wrow(j, _):
        pltpu.async_copy(srcf.at[pl.ds(j * 128, 128)],
                         srcc_hbm.at[cid].at[sid].at[j], sem)
        pltpu.async_copy(dstf.at[pl.ds(j * 128, 128)],
                         dstc_hbm.at[cid].at[sid].at[j], sem)
        return 0

    lax.fori_loop(0, nch, wrow, 0)

    def drow(j, _):
        pltpu.make_async_copy(srcf.at[pl.ds(j * 128, 128)],
                              srcc_hbm.at[cid].at[sid].at[j], sem).wait()
        pltpu.make_async_copy(dstf.at[pl.ds(j * 128, 128)],
                              dstc_hbm.at[cid].at[sid].at[j], sem).wait()
        return 0

    lax.fori_loop(0, nch, drow, 0)
    ncb[pl.ds(0, 16)] = jnp.broadcast_to(nch, (16,)).astype(jnp.int32)
    pltpu.sync_copy(ncb, nch_hbm.at[cid].at[sid])

    pltpu.sync_copy(srcc_hbm.at[cid].at[sid], srcb)
    pltpu.sync_copy(dstc_hbm.at[cid].at[sid], dstb)

    def dchunk(j, _):
        pltpu.async_copy(onesb, dego_sp.at[srcb.at[j]], sem, add=True)
        pltpu.async_copy(onesb, degi_sp.at[dstb.at[j]], sem, add=True)
        return 0

    lax.fori_loop(0, nch, dchunk, 0)

    def ddrain(j, _):
        pltpu.make_async_copy(onesb, dego_sp.at[srcb.at[0]], sem).wait()
        pltpu.make_async_copy(onesb, degi_sp.at[dstb.at[0]], sem).wait()
        return 0

    lax.fori_loop(0, nch, ddrain, 0)
    plsc.subcore_barrier()

    base = cid * (NP // 2) + sid * 320
    pltpu.sync_copy(dego_sp.at[pl.ds(base, 320)], dgb)
    pltpu.sync_copy(dgb, no_hbm.at[pl.ds(base, 320)])
    pltpu.sync_copy(degi_sp.at[pl.ds(base, 320)], dgb)
    pltpu.sync_copy(dgb, ni_hbm.at[pl.ds(base, 320)])


def _sc_compact(mask, src3, dst3):
    f = pl.kernel(
        _compact_body,
        out_type=[jax.ShapeDtypeStruct((NP,), jnp.float32),
                  jax.ShapeDtypeStruct((NP,), jnp.float32),
                  jax.ShapeDtypeStruct((2, 16, ECH, 128), jnp.int32),
                  jax.ShapeDtypeStruct((2, 16, ECH, 128), jnp.int32),
                  jax.ShapeDtypeStruct((2, 16, 16), jnp.int32)],
        mesh=_sc_mesh(),
        compiler_params=pltpu.CompilerParams(needs_layout_passes=False),
        scratch_types=[
            pltpu.VMEM((NP,), jnp.float32),
            pltpu.VMEM((ECH, 128), jnp.int32),
            pltpu.VMEM((ECH, 128), jnp.int32),
            pltpu.VMEM((EPT + 32,), jnp.int32),
            pltpu.VMEM((EPT + 32,), jnp.int32),
            pltpu.VMEM((128,), jnp.float32),
            pltpu.VMEM((16,), jnp.int32),
            pltpu.VMEM((NPT,), jnp.float32),
            pltpu.VMEM((320,), jnp.float32),
            pltpu.VMEM_SHARED((NP,), jnp.float32),
            pltpu.VMEM_SHARED((NP,), jnp.float32),
            pltpu.SemaphoreType.DMA,
        ],
    )
    return f(src3, dst3, mask)


NPA = N + 16


def _agg_body(h_hbm, src_hbm, dst_hbm, nch_hbm, out_hbm,
              sring, dring, ncb, gbuf, acc_sp, semi, semg, sems, semz):
    cid = lax.axis_index("c")
    sid = lax.axis_index("s")
    pltpu.sync_copy(nch_hbm.at[cid].at[sid], ncb)
    nch = ncb[pl.ds(0, 16)][0]
    _zero_vec(gbuf.at[0], 128)

    zb0 = sid * (NPA // 16)

    def zloop(t, _):
        pltpu.async_copy(gbuf.at[0], acc_sp.at[pl.ds(zb0 + t * 128, 128)],
                         semz)
        return 0

    lax.fori_loop(0, 4, zloop, 0)
    pltpu.async_copy(gbuf.at[0].at[pl.ds(0, NPA // 16 - 512)],
                     acc_sp.at[pl.ds(zb0 + 512, NPA // 16 - 512)], semz)

    def si(j):
        pltpu.async_copy(src_hbm.at[cid].at[sid].at[j], sring.at[j % 3],
                         semi)
        pltpu.async_copy(dst_hbm.at[cid].at[sid].at[j], dring.at[j % 3],
                         semi)

    def wi():
        pltpu.make_async_copy(src_hbm.at[cid].at[sid].at[0], sring.at[0],
                              semi).wait()
        pltpu.make_async_copy(dst_hbm.at[cid].at[sid].at[0], dring.at[0],
                              semi).wait()

    def sg(j):
        pltpu.async_copy(h_hbm.at[cid].at[sring.at[j % 3]], gbuf.at[j % 3],
                         semg)

    def wg():
        pltpu.make_async_copy(h_hbm.at[cid].at[sring.at[0]], gbuf.at[0],
                              semg).wait()

    def ss(j):
        pltpu.async_copy(gbuf.at[j % 3], acc_sp.at[dring.at[j % 3]], sems,
                         add=True)

    def ws():
        pltpu.make_async_copy(gbuf.at[0], acc_sp.at[dring.at[0]],
                              sems).wait()

    si(0)
    wi()
    sg(0)

    @pl.when(nch >= 2)
    def _():
        si(1)

    def zdrain(t, _):
        pltpu.make_async_copy(gbuf.at[0],
                              acc_sp.at[pl.ds(zb0, 128)], semz).wait()
        return 0

    lax.fori_loop(0, 4, zdrain, 0)
    pltpu.make_async_copy(gbuf.at[0].at[pl.ds(0, NPA // 16 - 512)],
                          acc_sp.at[pl.ds(zb0 + 512, NPA // 16 - 512)],
                          semz).wait()
    plsc.subcore_barrier()

    def chunk(j, _):
        wg()

        @pl.when(j + 1 < nch)
        def _():
            wi()

            @pl.when(j >= 2)
            def _():
                ws()

            sg(j + 1)

        ss(j)

        @pl.when(j + 2 < nch)
        def _():
            si(j + 2)

        return 0

    lax.fori_loop(0, nch, chunk, 0)

    @pl.when(nch >= 2)
    def _():
        ws()

    ws()
    plsc.subcore_barrier()

    _zero_vec(gbuf.at[0], 128)
    wb0 = sid * NPT

    @pl.when(sid < 15)
    def _():
        def wloop(t, _):
            rows = wb0 + t * 128
            buf = gbuf.at[1 + t % 2]
            pltpu.sync_copy(acc_sp.at[pl.ds(rows, 128)], buf)
            pltpu.async_copy(buf, out_hbm.at[cid].at[pl.ds(rows, 128)],
                             semg)
            return 0

        lax.fori_loop(0, NPT // 128, wloop, 0)

        def wdrain(t, _):
            pltpu.make_async_copy(gbuf.at[1],
                                  out_hbm.at[cid].at[pl.ds(wb0, 128)],
                                  semg).wait()
            return 0

        lax.fori_loop(0, NPT // 128, wdrain, 0)

    @pl.when(sid == 15)
    def _():
        def wloop(t, _):
            rows = 15 * NPT + t * 128
            pltpu.sync_copy(acc_sp.at[pl.ds(rows, 128)], gbuf.at[1])
            pltpu.sync_copy(gbuf.at[1], out_hbm.at[cid].at[pl.ds(rows, 128)])
            return 0

        lax.fori_loop(0, 3, wloop, 0)
        pltpu.sync_copy(acc_sp.at[pl.ds(9984, 32)],
                        gbuf.at[1].at[pl.ds(0, 32)])
        pltpu.sync_copy(gbuf.at[1].at[pl.ds(0, 32)],
                        out_hbm.at[cid].at[pl.ds(9984, 32)])
        pltpu.sync_copy(gbuf.at[0].at[pl.ds(0, 96)],
                        out_hbm.at[cid].at[pl.ds(NPA, 96)])
        pltpu.sync_copy(gbuf.at[0], out_hbm.at[cid].at[pl.ds(10112, 128)])


def _sc_aggregate(hsplit, srcc, dstc, nchs):
    f = pl.kernel(
        _agg_body,
        out_type=jax.ShapeDtypeStruct((2, NP, 128), jnp.float32),
        mesh=_sc_mesh(),
        compiler_params=pltpu.CompilerParams(needs_layout_passes=False),
        scratch_types=[
            pltpu.VMEM((3, 128), jnp.int32),
            pltpu.VMEM((3, 128), jnp.int32),
            pltpu.VMEM((16,), jnp.int32),
            pltpu.VMEM((3, 128, 128), jnp.float32),
            pltpu.VMEM_SHARED((NPA, 128), jnp.float32),
            pltpu.SemaphoreType.DMA,
            pltpu.SemaphoreType.DMA,
            pltpu.SemaphoreType.DMA,
            pltpu.SemaphoreType.DMA,
        ],
    )
    return f(hsplit, srcc, dstc, nchs)


def _aggs_body(s_hbm, src_hbm, dst_hbm, nch_hbm, out_hbm,
               s_v, srcb, dstb, ncb, sb, zb, agg_sp, sem):
    cid = lax.axis_index("c")
    sid = lax.axis_index("s")
    pltpu.sync_copy(s_hbm, s_v)
    pltpu.sync_copy(src_hbm.at[cid].at[sid], srcb)
    pltpu.sync_copy(dst_hbm.at[cid].at[sid], dstb)
    pltpu.sync_copy(nch_hbm.at[cid].at[sid], ncb)
    nch = ncb[pl.ds(0, 16)][0]
    _zero_flat(zb, NPT // 16)
    pltpu.sync_copy(zb, agg_sp.at[pl.ds(sid * NPT, NPT)])
    plsc.subcore_barrier()

    def chunk(j, _):
        for l in range(8):
            s16 = srcb[j, pl.ds(l * 16, 16)]
            sb[j, pl.ds(l * 16, 16)] = plsc.load_gather(
                s_v, [s16])
        pltpu.async_copy(sb.at[j], agg_sp.at[dstb.at[j]], sem, add=True)
        return 0

    lax.fori_loop(0, nch, chunk, 0)

    def sdrain(j, _):
        pltpu.make_async_copy(sb.at[0], agg_sp.at[dstb.at[0]], sem).wait()
        return 0

    lax.fori_loop(0, nch, sdrain, 0)
    plsc.subcore_barrier()
    base = cid * (NP // 2) + sid * 320
    pltpu.sync_copy(agg_sp.at[pl.ds(base, 320)], zb.at[pl.ds(0, 320)])
    pltpu.sync_copy(zb.at[pl.ds(0, 320)], out_hbm.at[pl.ds(base, 320)])


def _sc_aggregate_scalar(s_flat, srcc, dstc, nchs):
    f = pl.kernel(
        _aggs_body,
        out_type=jax.ShapeDtypeStruct((NP,), jnp.float32),
        mesh=_sc_mesh(),
        compiler_params=pltpu.CompilerParams(needs_layout_passes=False),
        scratch_types=[
            pltpu.VMEM((NP,), jnp.float32),
            pltpu.VMEM((ECH, 128), jnp.int32),
            pltpu.VMEM((ECH, 128), jnp.int32),
            pltpu.VMEM((16,), jnp.int32),
            pltpu.VMEM((ECH, 128), jnp.float32),
            pltpu.VMEM((NPT,), jnp.float32),
            pltpu.VMEM_SHARED((NP,), jnp.float32),
            pltpu.SemaphoreType.DMA,
        ],
    )
    return f(s_flat, srcc, dstc, nchs)


def kernel(x, convW0, convb0, poolW0, poolb0, convW1, convb1, poolW1, poolb1,
           convW2, convb2, poolW2, poolb2, lin1W, lin1b, lin2W, lin2b,
           lin3W, lin3b, edge_index):
    pad_idx = N + (jnp.arange(EP - E, dtype=jnp.int32) % (NP - N))
    src_pad = jnp.concatenate([edge_index[0], pad_idx])
    dst_pad = jnp.concatenate([edge_index[1], pad_idx])
    src3 = src_pad.reshape(16, ECH, 128)
    dst3 = dst_pad.reshape(16, ECH, 128)
    feat = jnp.pad(x, ((0, NP - N), (0, 0)))
    mask = (jnp.arange(NP) < N).astype(jnp.float32)
    mult = jnp.ones((NP, 1), jnp.float32)
    nmask = mask[:, None]

    layers = [(convW0, convb0, poolW0, poolb0),
              (convW1, convb1, poolW1, poolb1),
              (convW2, convb2, poolW2, poolb2)]
    ksizes = [5000, 2500, 1250]
    ros = []
    kprev = 1
    for (wc, bc, wp, bp), k in zip(layers, ksizes):
        dego, degi, srcc, dstc, nchs = _sc_compact(mask, src3, dst3)
        hs, ro_prev = _scaled_mm(feat, mult, nmask, dego[:, None], wc,
                                 kprev)
        ros.append(ro_prev)
        agg = _sc_aggregate(hs, srcc, dstc, nchs)
        feat, s_scaled = _epilogue(agg, degi[:, None], bc[None, :],
                                   mask[:, None], wp.reshape(1, D),
                                   dego[:, None])
        aggs = _sc_aggregate_scalar(s_scaled.reshape(NP), srcc, dstc,
                                    nchs)
        nm80, mult80 = _sag_topk(aggs.reshape(NP // 128, 128),
                                 degi.reshape(NP // 128, 128),
                                 bp.reshape(1, 1),
                                 mask.reshape(NP // 128, 128), k)
        nmask = nm80.reshape(NP, 1)
        mult = mult80.reshape(NP, 1)
        mask = nmask[:, 0]
        kprev = k
    ros.append(_apply_pool(feat, mult, nmask, 1250))
    ros = ros[1:]
    return _mlp(jnp.stack(ros), lin1W, lin1b[None, :], lin2W, lin2b[None, :],
                lin3W, lin3b[None, :])
